# Initial kernel scaffold; baseline (speedup 1.0000x reference)
import jax
import jax.numpy as jnp
from jax.experimental import pallas as pl


def kernel(probe_temperature, probe_locations, cell_adjacency, cell_to_pin_mapping, We1, be1, We2, be2, Wp1, bp1, Wp2, bp2, Wt1, bt1, Wt2, bt2):
    raise NotImplementedError("write your pallas kernel here")


# speedup vs baseline: 31.6138x; 31.6138x over previous
import functools

import jax
import jax.numpy as jnp
from jax import lax
from jax.experimental import pallas as pl
from jax.experimental.pallas import tpu as pltpu
from jax.experimental.pallas import tpu_sc as plsc

N = 50000
B = 4
PINS = 512
E = 800000
NP = 51200
SLC = NP // 16
PP = 640
TRASH = 512
NW = 32
NCHUNK = 196
EPT = NCHUNK * 128
EP = EPT * NW
PCH = NP // 8 // 128
BLK = 1024
GRID = NP // BLK

_mesh = plsc.VectorSubcoreMesh(core_axis_name="c", subcore_axis_name="s")
_f32 = jnp.float32


def _deg_body(dstw, ones_h, z_n, out, dstv, onesv, acc):
    cid = lax.axis_index("c")
    sid = lax.axis_index("s")
    wid = cid * 16 + sid
    r0 = sid * SLC
    pltpu.sync_copy(z_n.at[pl.ds(r0, SLC)], acc.at[pl.ds(r0, SLC)])
    pltpu.sync_copy(dstw.at[wid], dstv)
    pltpu.sync_copy(ones_h, onesv)
    plsc.subcore_barrier()

    @pl.loop(0, NCHUNK)
    def _chunk(j):
        pltpu.sync_copy(onesv, acc.at[dstv.at[j]], add=True)

    plsc.subcore_barrier()
    pltpu.sync_copy(acc.at[pl.ds(r0, SLC)], out.at[cid].at[pl.ds(r0, SLC)])


_deg = pl.kernel(
    _deg_body,
    out_type=jax.ShapeDtypeStruct((2, NP), _f32),
    mesh=_mesh,
    compiler_params=pltpu.CompilerParams(use_tc_tiling_on_sc=False),
    scratch_types=[
        pltpu.VMEM((NCHUNK, 128), jnp.int32),
        pltpu.VMEM((128,), _f32),
        pltpu.VMEM_SHARED((NP,), _f32),
    ],
)


def _agg_body(tbl, srcw, dstw, z16, out, srcv, dstv, rows, acc, sem):
    cid = lax.axis_index("c")
    sid = lax.axis_index("s")
    wid = cid * 16 + sid
    r0 = sid * SLC
    pltpu.sync_copy(z16.at[pl.ds(r0, SLC)], acc.at[pl.ds(r0, SLC)])
    pltpu.sync_copy(srcw.at[wid], srcv)
    pltpu.sync_copy(dstw.at[wid], dstv)
    plsc.subcore_barrier()

    @pl.loop(0, NCHUNK)
    def _chunk(j):
        pltpu.async_copy(tbl.at[srcv.at[j]], rows, sem).wait()
        pltpu.sync_copy(rows, acc.at[dstv.at[j]], add=True)

    plsc.subcore_barrier()
    pltpu.sync_copy(acc.at[pl.ds(r0, SLC)], out.at[cid].at[pl.ds(r0, SLC)])


_agg = pl.kernel(
    _agg_body,
    out_type=jax.ShapeDtypeStruct((2, NP, 16), _f32),
    mesh=_mesh,
    compiler_params=pltpu.CompilerParams(use_tc_tiling_on_sc=False),
    scratch_types=[
        pltpu.VMEM((NCHUNK, 128), jnp.int32),
        pltpu.VMEM((NCHUNK, 128), jnp.int32),
        pltpu.VMEM((128, 16), _f32),
        pltpu.VMEM_SHARED((NP, 16), _f32),
        pltpu.SemaphoreType.DMA,
    ],
)


def _pin_body(h2, r2, idxw, pww, zp16, zp, sums, cnts, idxv, pwv, rows, acc, cacc):
    cid = lax.axis_index("c")
    sid = lax.axis_index("s")
    b = sid % 4
    g = sid // 4
    gg = cid * 4 + g
    rowbase = gg * (NP // 8)

    @pl.when(g == 0)
    def _zero():
        pltpu.sync_copy(zp16, acc.at[b])

    @pl.when(sid == 0)
    def _zeroc():
        pltpu.sync_copy(zp, cacc)

    pltpu.sync_copy(idxw.at[gg], idxv)
    pltpu.sync_copy(pww.at[gg], pwv)
    plsc.subcore_barrier()

    @pl.loop(0, PCH)
    def _chunk(j):
        @pl.when(b == 0)
        def _g0():
            pltpu.sync_copy(h2.at[pl.ds(rowbase + j * 128, 128)], rows)
            pltpu.sync_copy(pwv.at[j], cacc.at[idxv.at[j]], add=True)

        @pl.when(b != 0)
        def _gr():
            pltpu.sync_copy(r2.at[b - 1].at[pl.ds(rowbase + j * 128, 128)], rows)

        pltpu.sync_copy(rows, acc.at[b].at[idxv.at[j]], add=True)

    plsc.subcore_barrier()

    @pl.when(g == 0)
    def _out():
        pltpu.sync_copy(acc.at[b], sums.at[cid].at[b])

    @pl.when(sid == 0)
    def _outc():
        pltpu.sync_copy(cacc, cnts.at[cid])


_pin = pl.kernel(
    _pin_body,
    out_type=(
        jax.ShapeDtypeStruct((2, B, PP, 16), _f32),
        jax.ShapeDtypeStruct((2, PP), _f32),
    ),
    mesh=_mesh,
    compiler_params=pltpu.CompilerParams(use_tc_tiling_on_sc=False),
    scratch_types=[
        pltpu.VMEM((PCH, 128), jnp.int32),
        pltpu.VMEM((PCH, 128), _f32),
        pltpu.VMEM((128, 16), _f32),
        pltpu.VMEM_SHARED((B, PP, 16), _f32),
        pltpu.VMEM_SHARED((PP,), _f32),
    ],
)


def _s1_body(degp0, degp1, pv0, pm, dinv_o, tbl_o):
    deg = degp0[...] + degp1[...] + 1.0
    dv = lax.rsqrt(deg)
    dinv_o[...] = dv
    ci = lax.broadcasted_iota(jnp.int32, (BLK, 16), 1)
    vals = jnp.where(ci == 0, pv0[...][:, None],
                     jnp.where(ci == 1, pm[...][:, None], 0.0))
    tbl_o[...] = dv[:, None] * vals


def _vec_spec():
    return pl.BlockSpec((BLK,), lambda i: (i,))


def _tbl_spec():
    return pl.BlockSpec((BLK, 16), lambda i: (i, 0))


def _w_spec(shape):
    return pl.BlockSpec(shape, lambda i: tuple(0 for _ in shape))


def _s1(degp, pv0, pm):
    return pl.pallas_call(
        _s1_body,
        grid=(GRID,),
        in_specs=[_vec_spec(), _vec_spec(), _vec_spec(), _vec_spec()],
        out_specs=[_vec_spec(), _tbl_spec()],
        out_shape=[
            jax.ShapeDtypeStruct((NP,), _f32),
            jax.ShapeDtypeStruct((NP, 16), _f32),
        ],
    )(degp[0], degp[1], pv0, pm)


def _make_stage(emit_h):
    def body(acc0, acc1, tblp, dinv, w, bb, *outs):
        dv = dinv[...]
        u = (acc0[...] + acc1[...] + tblp[...]) * dv[:, None]
        h = jnp.dot(u, w[...], preferred_element_type=_f32) + bb[...][None, :]
        h = jnp.maximum(h, 0.0)
        outs[0][...] = h * dv[:, None]
        if emit_h:
            outs[1][...] = h
    return body


def _stage(accp, tblp, dinv, w, bb, emit_h=False):
    n_out = 2 if emit_h else 1
    out_shape = [jax.ShapeDtypeStruct((NP, 16), _f32)] * n_out
    return pl.pallas_call(
        _make_stage(emit_h),
        grid=(GRID,),
        in_specs=[_tbl_spec(), _tbl_spec(), _tbl_spec(), _vec_spec(),
                  _w_spec((16, 16)), _w_spec((16,))],
        out_specs=[_tbl_spec()] * n_out,
        out_shape=out_shape,
    )(accp[0], accp[1], tblp, dinv, w, bb)


def _s5_body(acc0, acc1, tblp, dinv, w, bb, t_o):
    dv = dinv[...]
    u = (acc0[...] + acc1[...] + tblp[...]) * dv[:, None]
    t_o[...] = jnp.sum(u * w[...][None, :], axis=1) + bb[0]


def _s5(accp, tblp, dinv, wcol, bb):
    return pl.pallas_call(
        _s5_body,
        grid=(GRID,),
        in_specs=[_tbl_spec(), _tbl_spec(), _tbl_spec(), _vec_spec(),
                  _w_spec((16,)), _w_spec((1,))],
        out_specs=[_vec_spec()],
        out_shape=[jax.ShapeDtypeStruct((NP,), _f32)],
    )(accp[0], accp[1], tblp, dinv, wcol, bb)[0]


def _row_body(pvr, pm, we1a, we1b, be1, we2, be2, wt1, bt1, wt2, bt2,
              r2_o, tr_o):
    pv = pvr[0, 0]
    pmv = pm[...]
    h1 = pv[:, None] * we1a[...][None, :] + pmv[:, None] * we1b[...][None, :]
    h1 = jnp.maximum(h1 + be1[...][None, :], 0.0)
    h2 = jnp.maximum(
        jnp.dot(h1, we2[...], preferred_element_type=_f32) + be2[...][None, :], 0.0)
    r2_o[0] = h2
    h3 = jnp.maximum(
        jnp.dot(h2, wt1[...], preferred_element_type=_f32) + bt1[...][None, :], 0.0)
    tr_o[0, 0] = jnp.sum(h3 * wt2[...][None, :], axis=1) + bt2[0]


def _row(pvr, pm, we1, be1, we2, be2, wt1, bt1, wt2col, bt2):
    def vspec2():
        return pl.BlockSpec((1, 1, BLK), lambda b, i: (b, 0, i))

    def wspec(shape):
        return pl.BlockSpec(shape, lambda b, i: tuple(0 for _ in shape))

    return pl.pallas_call(
        _row_body,
        grid=(3, GRID),
        in_specs=[vspec2(), pl.BlockSpec((BLK,), lambda b, i: (i,)),
                  wspec((16,)), wspec((16,)), wspec((16,)),
                  wspec((16, 16)), wspec((16,)),
                  wspec((16, 16)), wspec((16,)),
                  wspec((16,)), wspec((1,))],
        out_specs=[pl.BlockSpec((1, BLK, 16), lambda b, i: (b, i, 0)), vspec2()],
        out_shape=[
            jax.ShapeDtypeStruct((3, NP, 16), _f32),
            jax.ShapeDtypeStruct((3, 1, NP), _f32),
        ],
    )(pvr.reshape(3, 1, NP), pm, we1[0], we1[1], be1, we2, be2, wt1, bt1,
      wt2col, bt2)


def _pinmlp_body(sf, recip, wp1, bp1, wp2, bp2, out_o):
    m = (sf[0] + sf[1]) * recip[...][None, :]
    g = jnp.dot(m, wp1[...], preferred_element_type=_f32) + bp1[...][None, :]
    g = jnp.maximum(g, 0.0)
    out_o[...] = jnp.dot(g, wp2[...], preferred_element_type=_f32) + bp2[...][None, :]


def _pinmlp(sf, recip, wp1, bp1, wp2, bp2):
    return pl.pallas_call(
        _pinmlp_body,
        out_shape=jax.ShapeDtypeStruct((B, PINS), _f32),
    )(sf, recip, wp1, bp1, wp2, bp2)


def kernel(probe_temperature, probe_locations, cell_adjacency, cell_to_pin_mapping,
           We1, be1, We2, be2, Wp1, bp1, Wp2, bp2, Wt1, bt1, Wt2, bt2):
    src = cell_adjacency[0]
    dst = cell_adjacency[1]
    pvp = jnp.zeros((B, NP), _f32).at[:, probe_locations].set(probe_temperature)
    pm = jnp.zeros((NP,), _f32).at[probe_locations].set(1.0)

    epad = jnp.full((EP - E,), NP - 1, jnp.int32)
    srcw = jnp.concatenate([src, epad]).reshape(NW, NCHUNK, 128)
    dstw = jnp.concatenate([dst, epad]).reshape(NW, NCHUNK, 128)

    valid = cell_to_pin_mapping >= 0
    idxp = jnp.where(valid, cell_to_pin_mapping, TRASH).astype(jnp.int32)
    idxw = jnp.concatenate(
        [idxp, jnp.full((NP - N,), TRASH, jnp.int32)]).reshape(8, PCH, 128)
    pww = jnp.concatenate(
        [valid.astype(_f32), jnp.zeros((NP - N,), _f32)]).reshape(8, PCH, 128)

    z16 = jnp.zeros((NP, 16), _f32)
    z_n = jnp.zeros((NP,), _f32)
    zp16 = jnp.zeros((PP, 16), _f32)
    zp = jnp.zeros((PP,), _f32)
    ones128 = jnp.ones((128,), _f32)

    degp = _deg(dstw, ones128, z_n)
    dinv, tbl1 = _s1(degp, pvp[0], pm)
    accp1 = _agg(tbl1, srcw, dstw, z16)
    (tbl2,) = _stage(accp1, tbl1, dinv, We1, be1)
    accp2 = _agg(tbl2, srcw, dstw, z16)
    tbl3, h2 = _stage(accp2, tbl2, dinv, We2, be2, emit_h=True)
    accp3 = _agg(tbl3, srcw, dstw, z16)
    (tbl4,) = _stage(accp3, tbl3, dinv, Wt1, bt1)
    accp4 = _agg(tbl4, srcw, dstw, z16)
    t0 = _s5(accp4, tbl4, dinv, Wt2[:, 0], bt2)

    r2, tr = _row(pvp[1:], pm, We1, be1, We2, be2, Wt1, bt1, Wt2[:, 0], bt2)
    tr = tr.reshape(3, NP)

    sumsp, cntsp = _pin(h2, r2, idxw, pww, zp16, zp)
    sf = sumsp[:, :, :TRASH, :].reshape(2, B, TRASH * 16)
    counts = cntsp[0] + cntsp[1]
    recip = jnp.repeat(1.0 / jnp.clip(counts[:TRASH], 1.0, None), 16)

    pin_power = _pinmlp(sf, recip, Wp1, bp1, Wp2, bp2)
    temperature = jnp.concatenate([t0[None, :N], tr[:, :N]], axis=0)
    return pin_power, temperature


# speedup vs baseline: 37.8769x; 1.1981x over previous
import functools

import jax
import jax.numpy as jnp
from jax import lax
from jax.experimental import pallas as pl
from jax.experimental.pallas import tpu as pltpu
from jax.experimental.pallas import tpu_sc as plsc

N = 50000
B = 4
PINS = 512
E = 800000
NP = 51200
SLC = NP // 16
PP = 640
TRASH = 512
NW = 32
NCHUNK = 196
EPT = NCHUNK * 128
EP = EPT * NW
PCH = NP // 8 // 128
BLK = 1024
GRID = NP // BLK

_mesh = plsc.VectorSubcoreMesh(core_axis_name="c", subcore_axis_name="s")
_f32 = jnp.float32


def _deg_body(dstw, ones_h, z_n, out, dstv, onesv, acc):
    cid = lax.axis_index("c")
    sid = lax.axis_index("s")
    wid = cid * 16 + sid
    r0 = sid * SLC
    pltpu.sync_copy(z_n.at[pl.ds(r0, SLC)], acc.at[pl.ds(r0, SLC)])
    pltpu.sync_copy(dstw.at[wid], dstv)
    pltpu.sync_copy(ones_h, onesv)
    plsc.subcore_barrier()

    @pl.loop(0, NCHUNK)
    def _chunk(j):
        pltpu.sync_copy(onesv, acc.at[dstv.at[j]], add=True)

    plsc.subcore_barrier()
    pltpu.sync_copy(acc.at[pl.ds(r0, SLC)], out.at[cid].at[pl.ds(r0, SLC)])


_deg = pl.kernel(
    _deg_body,
    out_type=jax.ShapeDtypeStruct((2, NP), _f32),
    mesh=_mesh,
    compiler_params=pltpu.CompilerParams(use_tc_tiling_on_sc=False),
    scratch_types=[
        pltpu.VMEM((NCHUNK, 128), jnp.int32),
        pltpu.VMEM((128,), _f32),
        pltpu.VMEM_SHARED((NP,), _f32),
    ],
)


def _agg_body(tbl, srcw, dstw, z16, out, srcv, dstv, rows0, rows1, acc,
              sem0, sem1):
    cid = lax.axis_index("c")
    sid = lax.axis_index("s")
    wid = cid * 16 + sid
    r0 = sid * SLC
    pltpu.sync_copy(z16.at[pl.ds(r0, SLC)], acc.at[pl.ds(r0, SLC)])
    pltpu.sync_copy(srcw.at[wid], srcv)
    pltpu.sync_copy(dstw.at[wid], dstv)
    plsc.subcore_barrier()

    pltpu.async_copy(tbl.at[srcv.at[0]], rows0, sem0)

    @pl.loop(0, NCHUNK // 2)
    def _chunk(k):
        j = 2 * k
        pltpu.async_copy(tbl.at[srcv.at[j + 1]], rows1, sem1)
        pltpu.make_async_copy(tbl.at[srcv.at[j]], rows0, sem0).wait()
        pltpu.sync_copy(rows0, acc.at[dstv.at[j]], add=True)

        @pl.when(k < NCHUNK // 2 - 1)
        def _next():
            pltpu.async_copy(tbl.at[srcv.at[j + 2]], rows0, sem0)

        pltpu.make_async_copy(tbl.at[srcv.at[j + 1]], rows1, sem1).wait()
        pltpu.sync_copy(rows1, acc.at[dstv.at[j + 1]], add=True)

    plsc.subcore_barrier()
    pltpu.sync_copy(acc.at[pl.ds(r0, SLC)], out.at[cid].at[pl.ds(r0, SLC)])


_agg = pl.kernel(
    _agg_body,
    out_type=jax.ShapeDtypeStruct((2, NP, 16), _f32),
    mesh=_mesh,
    compiler_params=pltpu.CompilerParams(use_tc_tiling_on_sc=False),
    scratch_types=[
        pltpu.VMEM((NCHUNK, 128), jnp.int32),
        pltpu.VMEM((NCHUNK, 128), jnp.int32),
        pltpu.VMEM((128, 16), _f32),
        pltpu.VMEM((128, 16), _f32),
        pltpu.VMEM_SHARED((NP, 16), _f32),
        pltpu.SemaphoreType.DMA,
        pltpu.SemaphoreType.DMA,
    ],
)


def _pin_body(h2, r2, idxw, pww, zp16, zp, sums, cnts, idxv, pwv, rows, acc, cacc):
    cid = lax.axis_index("c")
    sid = lax.axis_index("s")
    b = sid % 4
    g = sid // 4
    gg = cid * 4 + g
    rowbase = gg * (NP // 8)

    @pl.when(g == 0)
    def _zero():
        pltpu.sync_copy(zp16, acc.at[b])

    @pl.when(sid == 0)
    def _zeroc():
        pltpu.sync_copy(zp, cacc)

    pltpu.sync_copy(idxw.at[gg], idxv)
    pltpu.sync_copy(pww.at[gg], pwv)
    plsc.subcore_barrier()

    @pl.loop(0, PCH)
    def _chunk(j):
        @pl.when(b == 0)
        def _g0():
            pltpu.sync_copy(h2.at[pl.ds(rowbase + j * 128, 128)], rows)
            pltpu.sync_copy(pwv.at[j], cacc.at[idxv.at[j]], add=True)

        @pl.when(b != 0)
        def _gr():
            pltpu.sync_copy(r2.at[b - 1].at[pl.ds(rowbase + j * 128, 128)], rows)

        pltpu.sync_copy(rows, acc.at[b].at[idxv.at[j]], add=True)

    plsc.subcore_barrier()

    @pl.when(g == 0)
    def _out():
        pltpu.sync_copy(acc.at[b], sums.at[cid].at[b])

    @pl.when(sid == 0)
    def _outc():
        pltpu.sync_copy(cacc, cnts.at[cid])


_pin = pl.kernel(
    _pin_body,
    out_type=(
        jax.ShapeDtypeStruct((2, B, PP, 16), _f32),
        jax.ShapeDtypeStruct((2, PP), _f32),
    ),
    mesh=_mesh,
    compiler_params=pltpu.CompilerParams(use_tc_tiling_on_sc=False),
    scratch_types=[
        pltpu.VMEM((PCH, 128), jnp.int32),
        pltpu.VMEM((PCH, 128), _f32),
        pltpu.VMEM((128, 16), _f32),
        pltpu.VMEM_SHARED((B, PP, 16), _f32),
        pltpu.VMEM_SHARED((PP,), _f32),
    ],
)


def _s1_body(degp0, degp1, pv0, pm, dinv_o, tbl_o):
    deg = degp0[...] + degp1[...] + 1.0
    dv = lax.rsqrt(deg)
    dinv_o[...] = dv
    ci = lax.broadcasted_iota(jnp.int32, (BLK, 16), 1)
    vals = jnp.where(ci == 0, pv0[...][:, None],
                     jnp.where(ci == 1, pm[...][:, None], 0.0))
    tbl_o[...] = dv[:, None] * vals


def _vec_spec():
    return pl.BlockSpec((BLK,), lambda i: (i,))


def _tbl_spec():
    return pl.BlockSpec((BLK, 16), lambda i: (i, 0))


def _w_spec(shape):
    return pl.BlockSpec(shape, lambda i: tuple(0 for _ in shape))


def _s1(degp, pv0, pm):
    return pl.pallas_call(
        _s1_body,
        grid=(GRID,),
        in_specs=[_vec_spec(), _vec_spec(), _vec_spec(), _vec_spec()],
        out_specs=[_vec_spec(), _tbl_spec()],
        out_shape=[
            jax.ShapeDtypeStruct((NP,), _f32),
            jax.ShapeDtypeStruct((NP, 16), _f32),
        ],
    )(degp[0], degp[1], pv0, pm)


def _make_stage(emit_h):
    def body(acc0, acc1, tblp, dinv, w, bb, *outs):
        dv = dinv[...]
        u = (acc0[...] + acc1[...] + tblp[...]) * dv[:, None]
        h = jnp.dot(u, w[...], preferred_element_type=_f32) + bb[...][None, :]
        h = jnp.maximum(h, 0.0)
        outs[0][...] = h * dv[:, None]
        if emit_h:
            outs[1][...] = h
    return body


def _stage(accp, tblp, dinv, w, bb, emit_h=False):
    n_out = 2 if emit_h else 1
    out_shape = [jax.ShapeDtypeStruct((NP, 16), _f32)] * n_out
    return pl.pallas_call(
        _make_stage(emit_h),
        grid=(GRID,),
        in_specs=[_tbl_spec(), _tbl_spec(), _tbl_spec(), _vec_spec(),
                  _w_spec((16, 16)), _w_spec((16,))],
        out_specs=[_tbl_spec()] * n_out,
        out_shape=out_shape,
    )(accp[0], accp[1], tblp, dinv, w, bb)


def _s5_body(acc0, acc1, tblp, dinv, w, bb, t_o):
    dv = dinv[...]
    u = (acc0[...] + acc1[...] + tblp[...]) * dv[:, None]
    t_o[...] = jnp.sum(u * w[...][None, :], axis=1) + bb[0]


def _s5(accp, tblp, dinv, wcol, bb):
    return pl.pallas_call(
        _s5_body,
        grid=(GRID,),
        in_specs=[_tbl_spec(), _tbl_spec(), _tbl_spec(), _vec_spec(),
                  _w_spec((16,)), _w_spec((1,))],
        out_specs=[_vec_spec()],
        out_shape=[jax.ShapeDtypeStruct((NP,), _f32)],
    )(accp[0], accp[1], tblp, dinv, wcol, bb)[0]


def _row_body(pvr, pm, we1a, we1b, be1, we2, be2, wt1, bt1, wt2, bt2,
              r2_o, tr_o):
    pv = pvr[0, 0]
    pmv = pm[...]
    h1 = pv[:, None] * we1a[...][None, :] + pmv[:, None] * we1b[...][None, :]
    h1 = jnp.maximum(h1 + be1[...][None, :], 0.0)
    h2 = jnp.maximum(
        jnp.dot(h1, we2[...], preferred_element_type=_f32) + be2[...][None, :], 0.0)
    r2_o[0] = h2
    h3 = jnp.maximum(
        jnp.dot(h2, wt1[...], preferred_element_type=_f32) + bt1[...][None, :], 0.0)
    tr_o[0, 0] = jnp.sum(h3 * wt2[...][None, :], axis=1) + bt2[0]


def _row(pvr, pm, we1, be1, we2, be2, wt1, bt1, wt2col, bt2):
    def vspec2():
        return pl.BlockSpec((1, 1, BLK), lambda b, i: (b, 0, i))

    def wspec(shape):
        return pl.BlockSpec(shape, lambda b, i: tuple(0 for _ in shape))

    return pl.pallas_call(
        _row_body,
        grid=(3, GRID),
        in_specs=[vspec2(), pl.BlockSpec((BLK,), lambda b, i: (i,)),
                  wspec((16,)), wspec((16,)), wspec((16,)),
                  wspec((16, 16)), wspec((16,)),
                  wspec((16, 16)), wspec((16,)),
                  wspec((16,)), wspec((1,))],
        out_specs=[pl.BlockSpec((1, BLK, 16), lambda b, i: (b, i, 0)), vspec2()],
        out_shape=[
            jax.ShapeDtypeStruct((3, NP, 16), _f32),
            jax.ShapeDtypeStruct((3, 1, NP), _f32),
        ],
    )(pvr.reshape(3, 1, NP), pm, we1[0], we1[1], be1, we2, be2, wt1, bt1,
      wt2col, bt2)


def _pinmlp_body(sf, recip, wp1, bp1, wp2, bp2, out_o):
    m = (sf[0] + sf[1]) * recip[...][None, :]
    g = jnp.dot(m, wp1[...], preferred_element_type=_f32) + bp1[...][None, :]
    g = jnp.maximum(g, 0.0)
    out_o[...] = jnp.dot(g, wp2[...], preferred_element_type=_f32) + bp2[...][None, :]


def _pinmlp(sf, recip, wp1, bp1, wp2, bp2):
    return pl.pallas_call(
        _pinmlp_body,
        out_shape=jax.ShapeDtypeStruct((B, PINS), _f32),
    )(sf, recip, wp1, bp1, wp2, bp2)


def kernel(probe_temperature, probe_locations, cell_adjacency, cell_to_pin_mapping,
           We1, be1, We2, be2, Wp1, bp1, Wp2, bp2, Wt1, bt1, Wt2, bt2):
    src = cell_adjacency[0]
    dst = cell_adjacency[1]
    pvp = jnp.zeros((B, NP), _f32).at[:, probe_locations].set(probe_temperature)
    pm = jnp.zeros((NP,), _f32).at[probe_locations].set(1.0)

    epad = jnp.full((EP - E,), NP - 1, jnp.int32)
    srcw = jnp.concatenate([src, epad]).reshape(NW, NCHUNK, 128)
    dstw = jnp.concatenate([dst, epad]).reshape(NW, NCHUNK, 128)

    valid = cell_to_pin_mapping >= 0
    idxp = jnp.where(valid, cell_to_pin_mapping, TRASH).astype(jnp.int32)
    idxw = jnp.concatenate(
        [idxp, jnp.full((NP - N,), TRASH, jnp.int32)]).reshape(8, PCH, 128)
    pww = jnp.concatenate(
        [valid.astype(_f32), jnp.zeros((NP - N,), _f32)]).reshape(8, PCH, 128)

    z16 = jnp.zeros((NP, 16), _f32)
    z_n = jnp.zeros((NP,), _f32)
    zp16 = jnp.zeros((PP, 16), _f32)
    zp = jnp.zeros((PP,), _f32)
    ones128 = jnp.ones((128,), _f32)

    degp = _deg(dstw, ones128, z_n)
    dinv, tbl1 = _s1(degp, pvp[0], pm)
    accp1 = _agg(tbl1, srcw, dstw, z16)
    (tbl2,) = _stage(accp1, tbl1, dinv, We1, be1)
    accp2 = _agg(tbl2, srcw, dstw, z16)
    tbl3, h2 = _stage(accp2, tbl2, dinv, We2, be2, emit_h=True)
    accp3 = _agg(tbl3, srcw, dstw, z16)
    (tbl4,) = _stage(accp3, tbl3, dinv, Wt1, bt1)
    accp4 = _agg(tbl4, srcw, dstw, z16)
    t0 = _s5(accp4, tbl4, dinv, Wt2[:, 0], bt2)

    r2, tr = _row(pvp[1:], pm, We1, be1, We2, be2, Wt1, bt1, Wt2[:, 0], bt2)
    tr = tr.reshape(3, NP)

    sumsp, cntsp = _pin(h2, r2, idxw, pww, zp16, zp)
    sf = sumsp[:, :, :TRASH, :].reshape(2, B, TRASH * 16)
    counts = cntsp[0] + cntsp[1]
    recip = jnp.repeat(1.0 / jnp.clip(counts[:TRASH], 1.0, None), 16)

    pin_power = _pinmlp(sf, recip, Wp1, bp1, Wp2, bp2)
    temperature = jnp.concatenate([t0[None, :N], tr[:, :N]], axis=0)
    return pin_power, temperature


# speedup vs baseline: 46.4837x; 1.2272x over previous
import jax
import jax.numpy as jnp
from jax import lax
from jax.experimental import pallas as pl
from jax.experimental.pallas import tpu as pltpu
from jax.experimental.pallas import tpu_sc as plsc

N = 50000
B = 4
PINS = 512
E = 800000
NP = 51200
SLC = NP // 16
PP = 640
TRASH = 512
NW = 32
NCHUNK = 196
EPT = NCHUNK * 128
EP = EPT * NW
PCH = NP // 8 // 128
NF = NP // 8
FBLK = 256
FGRID = NF // FBLK

_mesh = plsc.VectorSubcoreMesh(core_axis_name="c", subcore_axis_name="s")
_f32 = jnp.float32


def _deg_body(dstw, ones_h, z16, out, dstv, onesv, acc):
    cid = lax.axis_index("c")
    sid = lax.axis_index("s")
    wid = cid * 16 + sid
    r0 = sid * SLC
    pltpu.sync_copy(z16.at[pl.ds(r0, SLC)], acc.at[pl.ds(r0, SLC)])
    pltpu.sync_copy(dstw.at[wid], dstv)
    pltpu.sync_copy(ones_h, onesv)
    plsc.subcore_barrier()

    @pl.loop(0, NCHUNK)
    def _chunk(j):
        pltpu.sync_copy(onesv, acc.at[dstv.at[j]], add=True)

    plsc.subcore_barrier()
    pltpu.sync_copy(acc.at[pl.ds(r0, SLC)], out.at[cid].at[pl.ds(r0, SLC)])


_deg = pl.kernel(
    _deg_body,
    out_type=jax.ShapeDtypeStruct((2, NP, 16), _f32),
    mesh=_mesh,
    compiler_params=pltpu.CompilerParams(use_tc_tiling_on_sc=False),
    scratch_types=[
        pltpu.VMEM((NCHUNK, 128), jnp.int32),
        pltpu.VMEM((128, 16), _f32),
        pltpu.VMEM_SHARED((NP, 16), _f32),
    ],
)


def _agg_body(tbl, srcw, dstw, z16, out, srcv, dstv, rows0, rows1, acc,
              sem0, sem1):
    cid = lax.axis_index("c")
    sid = lax.axis_index("s")
    wid = cid * 16 + sid
    r0 = sid * SLC
    pltpu.sync_copy(z16.at[pl.ds(r0, SLC)], acc.at[pl.ds(r0, SLC)])
    pltpu.sync_copy(srcw.at[wid], srcv)
    pltpu.sync_copy(dstw.at[wid], dstv)
    plsc.subcore_barrier()

    pltpu.async_copy(tbl.at[srcv.at[0]], rows0, sem0)

    @pl.loop(0, NCHUNK // 2)
    def _chunk(k):
        j = 2 * k
        pltpu.async_copy(tbl.at[srcv.at[j + 1]], rows1, sem1)
        pltpu.make_async_copy(tbl.at[srcv.at[j]], rows0, sem0).wait()
        pltpu.sync_copy(rows0, acc.at[dstv.at[j]], add=True)

        @pl.when(k < NCHUNK // 2 - 1)
        def _next():
            pltpu.async_copy(tbl.at[srcv.at[j + 2]], rows0, sem0)

        pltpu.make_async_copy(tbl.at[srcv.at[j + 1]], rows1, sem1).wait()
        pltpu.sync_copy(rows1, acc.at[dstv.at[j + 1]], add=True)

    plsc.subcore_barrier()
    pltpu.sync_copy(acc.at[pl.ds(r0, SLC)], out.at[cid].at[pl.ds(r0, SLC)])


_agg = pl.kernel(
    _agg_body,
    out_type=jax.ShapeDtypeStruct((2, NP, 16), _f32),
    mesh=_mesh,
    compiler_params=pltpu.CompilerParams(use_tc_tiling_on_sc=False),
    scratch_types=[
        pltpu.VMEM((NCHUNK, 128), jnp.int32),
        pltpu.VMEM((NCHUNK, 128), jnp.int32),
        pltpu.VMEM((128, 16), _f32),
        pltpu.VMEM((128, 16), _f32),
        pltpu.VMEM_SHARED((NP, 16), _f32),
        pltpu.SemaphoreType.DMA,
        pltpu.SemaphoreType.DMA,
    ],
)


def _pin_body(h2, r2, idxw, pww, zp16, zp, sums, cnts, idxv, pwv, rows, acc, cacc):
    cid = lax.axis_index("c")
    sid = lax.axis_index("s")
    b = sid % 4
    g = sid // 4
    gg = cid * 4 + g
    rowbase = gg * (NP // 8)

    @pl.when(g == 0)
    def _zero():
        pltpu.sync_copy(zp16, acc.at[b])

    @pl.when(sid == 0)
    def _zeroc():
        pltpu.sync_copy(zp, cacc)

    pltpu.sync_copy(idxw.at[gg], idxv)
    pltpu.sync_copy(pww.at[gg], pwv)
    plsc.subcore_barrier()

    @pl.loop(0, PCH)
    def _chunk(j):
        @pl.when(b == 0)
        def _g0():
            pltpu.sync_copy(h2.at[pl.ds(rowbase + j * 128, 128)], rows)
            pltpu.sync_copy(pwv.at[j], cacc.at[idxv.at[j]], add=True)

        @pl.when(b != 0)
        def _gr():
            pltpu.sync_copy(r2.at[b - 1].at[pl.ds(rowbase + j * 128, 128)], rows)

        pltpu.sync_copy(rows, acc.at[b].at[idxv.at[j]], add=True)

    plsc.subcore_barrier()

    @pl.when(g == 0)
    def _out():
        pltpu.sync_copy(acc.at[b], sums.at[cid].at[b])

    @pl.when(sid == 0)
    def _outc():
        pltpu.sync_copy(cacc, cnts.at[cid])


_pin = pl.kernel(
    _pin_body,
    out_type=(
        jax.ShapeDtypeStruct((2, B, PP, 16), _f32),
        jax.ShapeDtypeStruct((2, PP), _f32),
    ),
    mesh=_mesh,
    compiler_params=pltpu.CompilerParams(use_tc_tiling_on_sc=False),
    scratch_types=[
        pltpu.VMEM((PCH, 128), jnp.int32),
        pltpu.VMEM((PCH, 128), _f32),
        pltpu.VMEM((128, 16), _f32),
        pltpu.VMEM_SHARED((B, PP, 16), _f32),
        pltpu.VMEM_SHARED((PP,), _f32),
    ],
)


def _fspec():
    return pl.BlockSpec((FBLK, 128), lambda i: (i, 0))


def _wspec(shape):
    return pl.BlockSpec(shape, lambda i: tuple(0 for _ in shape))


def _s1_body(degf0, degf1, x0f, dinv_o, tbl_o):
    dv = lax.rsqrt(degf0[...] + degf1[...] + 1.0)
    dinv_o[...] = dv
    tbl_o[...] = dv * x0f[...]


def _s1(degf, x0f):
    return pl.pallas_call(
        _s1_body,
        grid=(FGRID,),
        in_specs=[_fspec(), _fspec(), _fspec()],
        out_specs=[_fspec(), _fspec()],
        out_shape=[
            jax.ShapeDtypeStruct((NF, 128), _f32),
            jax.ShapeDtypeStruct((NF, 128), _f32),
        ],
    )(degf[0], degf[1], x0f)


def _make_stage(emit_h):
    def body(acc0, acc1, tblp, dinv, w, bb, *outs):
        dv = dinv[...]
        u = (acc0[...] + acc1[...] + tblp[...]) * dv
        h = jnp.dot(u, w[...], preferred_element_type=_f32) + bb[...][None, :]
        h = jnp.maximum(h, 0.0)
        outs[0][...] = h * dv
        if emit_h:
            outs[1][...] = h
    return body


def _stage(accf, tblf, dinvf, wbig, bflat, emit_h=False):
    n_out = 2 if emit_h else 1
    out_shape = [jax.ShapeDtypeStruct((NF, 128), _f32)] * n_out
    return pl.pallas_call(
        _make_stage(emit_h),
        grid=(FGRID,),
        in_specs=[_fspec(), _fspec(), _fspec(), _fspec(),
                  _wspec((128, 128)), _wspec((128,))],
        out_specs=[_fspec()] * n_out,
        out_shape=out_shape,
    )(accf[0], accf[1], tblf, dinvf, wbig, bflat)


def _s5_body(acc0, acc1, tblp, dinv, w, bb, t_o):
    u = (acc0[...] + acc1[...] + tblp[...]) * dinv[...]
    t_o[...] = jnp.dot(u, w[...], preferred_element_type=_f32) + bb[0]


def _s5(accf, tblf, dinvf, wt2big, bt2):
    return pl.pallas_call(
        _s5_body,
        grid=(FGRID,),
        in_specs=[_fspec(), _fspec(), _fspec(), _fspec(),
                  _wspec((128, 8)), _wspec((1,))],
        out_specs=[pl.BlockSpec((FBLK, 8), lambda i: (i, 0))],
        out_shape=[jax.ShapeDtypeStruct((NF, 8), _f32)],
    )(accf[0], accf[1], tblf, dinvf, wt2big, bt2)[0]


def _row_body(xf, w1, b1, w2, b2, w3, b3, w4, b4, r2_o, tr_o):
    h1 = jnp.maximum(
        jnp.dot(xf[0], w1[...], preferred_element_type=_f32) + b1[...][None, :], 0.0)
    h2 = jnp.maximum(
        jnp.dot(h1, w2[...], preferred_element_type=_f32) + b2[...][None, :], 0.0)
    r2_o[0] = h2
    h3 = jnp.maximum(
        jnp.dot(h2, w3[...], preferred_element_type=_f32) + b3[...][None, :], 0.0)
    tr_o[0] = jnp.dot(h3, w4[...], preferred_element_type=_f32) + b4[0]


def _row(xrf, w1big, be1f, w2big, be2f, w3big, bt1f, wt2big, bt2):
    def bspec(minor):
        return pl.BlockSpec((1, FBLK, minor), lambda b, i: (b, i, 0))

    def wspec(shape):
        return pl.BlockSpec(shape, lambda b, i: tuple(0 for _ in shape))

    return pl.pallas_call(
        _row_body,
        grid=(3, FGRID),
        in_specs=[bspec(128),
                  wspec((128, 128)), wspec((128,)),
                  wspec((128, 128)), wspec((128,)),
                  wspec((128, 128)), wspec((128,)),
                  wspec((128, 8)), wspec((1,))],
        out_specs=[bspec(128), bspec(8)],
        out_shape=[
            jax.ShapeDtypeStruct((3, NF, 128), _f32),
            jax.ShapeDtypeStruct((3, NF, 8), _f32),
        ],
    )(xrf, w1big, be1f, w2big, be2f, w3big, bt1f, wt2big, bt2)


def _pinmlp_body(sf, recip, wp1, bp1, wp2, bp2, out_o):
    m = (sf[0] + sf[1]) * recip[...][None, :]
    g = jnp.dot(m, wp1[...], preferred_element_type=_f32) + bp1[...][None, :]
    g = jnp.maximum(g, 0.0)
    out_o[...] = jnp.dot(g, wp2[...], preferred_element_type=_f32) + bp2[...][None, :]


def _pinmlp(sf, recip, wp1, bp1, wp2, bp2):
    return pl.pallas_call(
        _pinmlp_body,
        out_shape=jax.ShapeDtypeStruct((B, PINS), _f32),
    )(sf, recip, wp1, bp1, wp2, bp2)


def _kron8(w):
    return jnp.kron(jnp.eye(8, dtype=_f32), w.astype(_f32))


def kernel(probe_temperature, probe_locations, cell_adjacency, cell_to_pin_mapping,
           We1, be1, We2, be2, Wp1, bp1, Wp2, bp2, Wt1, bt1, Wt2, bt2):
    src = cell_adjacency[0]
    dst = cell_adjacency[1]
    pvp = jnp.zeros((B, NP), _f32).at[:, probe_locations].set(probe_temperature)
    pm = jnp.zeros((NP,), _f32).at[probe_locations].set(1.0)

    epad = jnp.full((EP - E,), NP - 1, jnp.int32)
    srcw = jnp.concatenate([src, epad]).reshape(NW, NCHUNK, 128)
    dstw = jnp.concatenate([dst, epad]).reshape(NW, NCHUNK, 128)

    valid = cell_to_pin_mapping >= 0
    idxp = jnp.where(valid, cell_to_pin_mapping, TRASH).astype(jnp.int32)
    idxw = jnp.concatenate(
        [idxp, jnp.full((NP - N,), TRASH, jnp.int32)]).reshape(8, PCH, 128)
    pww = jnp.concatenate(
        [valid.astype(_f32), jnp.zeros((NP - N,), _f32)]).reshape(8, PCH, 128)

    z16 = jnp.zeros((NP, 16), _f32)
    zp16 = jnp.zeros((PP, 16), _f32)
    zp = jnp.zeros((PP,), _f32)
    ones16 = jnp.ones((128, 16), _f32)

    col = jnp.arange(128)
    csel = col // 16
    pv0r = jnp.take(pvp[0].reshape(NF, 8), csel, axis=1)
    pmr = jnp.take(pm.reshape(NF, 8), csel, axis=1)
    x0f = jnp.where((col % 16) == 0, pv0r, jnp.where((col % 16) == 1, pmr, 0.0))
    pvrr = jnp.take(pvp[1:].reshape(3, NF, 8), csel, axis=2)
    pmr3 = jnp.broadcast_to(pmr[None], (3, NF, 128))
    xrf = jnp.where((col % 16) == 0, pvrr,
                    jnp.where((col % 16) == 1, pmr3, 0.0))

    w1big = _kron8(jnp.zeros((16, 16), _f32).at[:2, :].set(We1))
    w2big = _kron8(We2)
    w3big = _kron8(Wt1)
    wt2big = _kron8(Wt2)
    be1f = jnp.tile(be1, 8)
    be2f = jnp.tile(be2, 8)
    bt1f = jnp.tile(bt1, 8)

    degp = _deg(dstw, ones16, z16)
    degf = degp.reshape(2, NF, 128)
    dinvf, tbl1f = _s1(degf, x0f)

    accf1 = _agg(tbl1f.reshape(NP, 16), srcw, dstw, z16).reshape(2, NF, 128)
    (tbl2f,) = _stage(accf1, tbl1f, dinvf, w1big, be1f)
    accf2 = _agg(tbl2f.reshape(NP, 16), srcw, dstw, z16).reshape(2, NF, 128)
    tbl3f, h2f = _stage(accf2, tbl2f, dinvf, w2big, be2f, emit_h=True)
    accf3 = _agg(tbl3f.reshape(NP, 16), srcw, dstw, z16).reshape(2, NF, 128)
    (tbl4f,) = _stage(accf3, tbl3f, dinvf, w3big, bt1f)
    accf4 = _agg(tbl4f.reshape(NP, 16), srcw, dstw, z16).reshape(2, NF, 128)
    t0f = _s5(accf4, tbl4f, dinvf, wt2big, bt2)

    r2f, trf = _row(xrf, w1big, be1f, w2big, be2f, w3big, bt1f, wt2big, bt2)

    sumsp, cntsp = _pin(h2f.reshape(NP, 16), r2f.reshape(3, NP, 16),
                        idxw, pww, zp16, zp)
    sf = sumsp.reshape(2, B, PP * 16)[:, :, :TRASH * 16]
    counts = cntsp[0] + cntsp[1]
    recip = jnp.repeat(1.0 / jnp.clip(counts[:TRASH], 1.0, None), 16)

    pin_power = _pinmlp(sf, recip, Wp1, bp1, Wp2, bp2)
    temperature = jnp.concatenate(
        [t0f.reshape(NP)[None, :N], trf.reshape(3, NP)[:, :N]], axis=0)
    return pin_power, temperature


# speedup vs baseline: 76.5005x; 1.6458x over previous
import jax
import jax.numpy as jnp
from jax import lax
from jax.experimental import pallas as pl
from jax.experimental.pallas import tpu as pltpu
from jax.experimental.pallas import tpu_sc as plsc

N = 50000
B = 4
PINS = 512
E = 800000
NP = 51200
SLC = NP // 16
PP = 640
TRASH = 512
NW = 32
NCHUNK = 196
EPT = NCHUNK * 128
EP = EPT * NW
PCH = NP // 8 // 128
NF = NP // 8
FBLK = 256
FGRID = NF // FBLK

_mesh = plsc.VectorSubcoreMesh(core_axis_name="c", subcore_axis_name="s")
_f32 = jnp.float32


def _deg_body(dstw, ones_h, z16, out, dstv, onesv, acc):
    cid = lax.axis_index("c")
    sid = lax.axis_index("s")
    wid = cid * 16 + sid
    r0 = sid * SLC
    pltpu.sync_copy(z16.at[pl.ds(r0, SLC)], acc.at[pl.ds(r0, SLC)])
    pltpu.sync_copy(dstw.at[wid], dstv)
    pltpu.sync_copy(ones_h, onesv)
    plsc.subcore_barrier()

    @pl.loop(0, NCHUNK)
    def _chunk(j):
        pltpu.sync_copy(onesv, acc.at[dstv.at[j]], add=True)

    plsc.subcore_barrier()
    pltpu.sync_copy(acc.at[pl.ds(r0, SLC)], out.at[cid].at[pl.ds(r0, SLC)])


_deg = pl.kernel(
    _deg_body,
    out_type=jax.ShapeDtypeStruct((2, NP, 16), _f32),
    mesh=_mesh,
    compiler_params=pltpu.CompilerParams(use_tc_tiling_on_sc=False),
    scratch_types=[
        pltpu.VMEM((NCHUNK, 128), jnp.int32),
        pltpu.VMEM((128, 16), _f32),
        pltpu.VMEM_SHARED((NP, 16), _f32),
    ],
)


def _agg_body(tbl, srcw, dstw, z16, out, srcv, dstv, r0b, r1b, r2b, r3b, acc,
              sg0, sg1, sg2, sg3, ss0, ss1, ss2, ss3):
    rowsb = [r0b, r1b, r2b, r3b]
    semg = [sg0, sg1, sg2, sg3]
    sems = [ss0, ss1, ss2, ss3]
    cid = lax.axis_index("c")
    sid = lax.axis_index("s")
    wid = cid * 16 + sid
    r0 = sid * SLC
    pltpu.sync_copy(z16.at[pl.ds(r0, SLC)], acc.at[pl.ds(r0, SLC)])
    pltpu.sync_copy(srcw.at[wid], srcv)
    pltpu.sync_copy(dstw.at[wid], dstv)
    plsc.subcore_barrier()

    for b in range(4):
        pltpu.async_copy(tbl.at[srcv.at[b]], rowsb[b], semg[b])

    @pl.loop(0, NCHUNK // 4)
    def _grp(k):
        for b in range(4):
            j = 4 * k + b
            pltpu.make_async_copy(tbl.at[srcv.at[j]], rowsb[b], semg[b]).wait()
            pltpu.async_copy(rowsb[b], acc.at[dstv.at[j]], sems[b], add=True)

            @pl.when(k < NCHUNK // 4 - 1)
            def _refill():
                pltpu.make_async_copy(rowsb[b], acc.at[dstv.at[j]],
                                      sems[b]).wait()
                pltpu.async_copy(tbl.at[srcv.at[j + 4]], rowsb[b], semg[b])

    for b in range(4):
        j = NCHUNK - 4 + b
        pltpu.make_async_copy(rowsb[b], acc.at[dstv.at[j]], sems[b]).wait()

    plsc.subcore_barrier()
    pltpu.sync_copy(acc.at[pl.ds(r0, SLC)], out.at[cid].at[pl.ds(r0, SLC)])


_agg = pl.kernel(
    _agg_body,
    out_type=jax.ShapeDtypeStruct((2, NP, 16), _f32),
    mesh=_mesh,
    compiler_params=pltpu.CompilerParams(use_tc_tiling_on_sc=False),
    scratch_types=[
        pltpu.VMEM((NCHUNK, 128), jnp.int32),
        pltpu.VMEM((NCHUNK, 128), jnp.int32),
        pltpu.VMEM((128, 16), _f32),
        pltpu.VMEM((128, 16), _f32),
        pltpu.VMEM((128, 16), _f32),
        pltpu.VMEM((128, 16), _f32),
        pltpu.VMEM_SHARED((NP, 16), _f32),
        pltpu.SemaphoreType.DMA,
        pltpu.SemaphoreType.DMA,
        pltpu.SemaphoreType.DMA,
        pltpu.SemaphoreType.DMA,
        pltpu.SemaphoreType.DMA,
        pltpu.SemaphoreType.DMA,
        pltpu.SemaphoreType.DMA,
        pltpu.SemaphoreType.DMA,
    ],
)


def _pin_body(h2, r2, idxw, pww, zp16, zp, sums, cnts, idxv, pwv,
              rows0, rows1, acc, cacc, sg0, sg1, ss0, ss1):
    rowsb = [rows0, rows1]
    semg = [sg0, sg1]
    sems = [ss0, ss1]
    cid = lax.axis_index("c")
    sid = lax.axis_index("s")
    b = sid % 4
    g = sid // 4
    gg = cid * 4 + g
    rowbase = gg * (NP // 8)

    @pl.when(g == 0)
    def _zero():
        pltpu.sync_copy(zp16, acc.at[b])

    @pl.when(sid == 0)
    def _zeroc():
        pltpu.sync_copy(zp, cacc)

    pltpu.sync_copy(idxw.at[gg], idxv)
    pltpu.sync_copy(pww.at[gg], pwv)
    plsc.subcore_barrier()

    def _load(j, rb, sg):
        @pl.when(b == 0)
        def _g0():
            pltpu.async_copy(h2.at[pl.ds(rowbase + j * 128, 128)], rb, sg)

        @pl.when(b != 0)
        def _gr():
            pltpu.async_copy(r2.at[b - 1].at[pl.ds(rowbase + j * 128, 128)],
                             rb, sg)

    def _wait_load(j, rb, sg):
        pltpu.make_async_copy(h2.at[pl.ds(rowbase + j * 128, 128)],
                              rb, sg).wait()

    for bb in range(2):
        _load(bb, rowsb[bb], semg[bb])

    @pl.loop(0, PCH // 2)
    def _grp(k):
        for bb in range(2):
            j = 2 * k + bb
            _wait_load(j, rowsb[bb], semg[bb])
            pltpu.async_copy(rowsb[bb], acc.at[b].at[idxv.at[j]],
                             sems[bb], add=True)

            @pl.when(b == 0)
            def _cnt():
                pltpu.sync_copy(pwv.at[j], cacc.at[idxv.at[j]], add=True)

            @pl.when(k < PCH // 2 - 1)
            def _refill():
                pltpu.make_async_copy(rowsb[bb], acc.at[b].at[idxv.at[j]],
                                      sems[bb]).wait()
                _load(j + 2, rowsb[bb], semg[bb])

    for bb in range(2):
        j = PCH - 2 + bb
        pltpu.make_async_copy(rowsb[bb], acc.at[b].at[idxv.at[j]],
                              sems[bb]).wait()

    plsc.subcore_barrier()

    @pl.when(g == 0)
    def _out():
        pltpu.sync_copy(acc.at[b], sums.at[cid].at[b])

    @pl.when(sid == 0)
    def _outc():
        pltpu.sync_copy(cacc, cnts.at[cid])


_pin = pl.kernel(
    _pin_body,
    out_type=(
        jax.ShapeDtypeStruct((2, B, PP, 16), _f32),
        jax.ShapeDtypeStruct((2, PP), _f32),
    ),
    mesh=_mesh,
    compiler_params=pltpu.CompilerParams(use_tc_tiling_on_sc=False),
    scratch_types=[
        pltpu.VMEM((PCH, 128), jnp.int32),
        pltpu.VMEM((PCH, 128), _f32),
        pltpu.VMEM((128, 16), _f32),
        pltpu.VMEM((128, 16), _f32),
        pltpu.VMEM_SHARED((B, PP, 16), _f32),
        pltpu.VMEM_SHARED((PP,), _f32),
        pltpu.SemaphoreType.DMA,
        pltpu.SemaphoreType.DMA,
        pltpu.SemaphoreType.DMA,
        pltpu.SemaphoreType.DMA,
    ],
)


def _fspec():
    return pl.BlockSpec((FBLK, 128), lambda i: (i, 0))


def _wspec(shape):
    return pl.BlockSpec(shape, lambda i: tuple(0 for _ in shape))


def _s1_body(degf0, degf1, x0f, dinv_o, tbl_o):
    dv = lax.rsqrt(degf0[...] + degf1[...] + 1.0)
    dinv_o[...] = dv
    tbl_o[...] = dv * x0f[...]


def _s1(degf, x0f):
    return pl.pallas_call(
        _s1_body,
        grid=(FGRID,),
        in_specs=[_fspec(), _fspec(), _fspec()],
        out_specs=[_fspec(), _fspec()],
        out_shape=[
            jax.ShapeDtypeStruct((NF, 128), _f32),
            jax.ShapeDtypeStruct((NF, 128), _f32),
        ],
    )(degf[0], degf[1], x0f)


def _accspec():
    return pl.BlockSpec((2, FBLK, 128), lambda i: (0, i, 0))


def _make_stage(emit_h):
    def body(accp, tblp, dinv, w, bb, *outs):
        dv = dinv[...]
        u = (accp[0] + accp[1] + tblp[...]) * dv
        h = jnp.dot(u, w[...], preferred_element_type=_f32) + bb[...][None, :]
        h = jnp.maximum(h, 0.0)
        outs[0][...] = h * dv
        if emit_h:
            outs[1][...] = h
    return body


def _stage(accf, tblf, dinvf, wbig, bflat, emit_h=False):
    n_out = 2 if emit_h else 1
    out_shape = [jax.ShapeDtypeStruct((NF, 128), _f32)] * n_out
    return pl.pallas_call(
        _make_stage(emit_h),
        grid=(FGRID,),
        in_specs=[_accspec(), _fspec(), _fspec(),
                  _wspec((128, 128)), _wspec((128,))],
        out_specs=[_fspec()] * n_out,
        out_shape=out_shape,
    )(accf, tblf, dinvf, wbig, bflat)


def _s5_body(accp, tblp, dinv, w, bb, t_o):
    u = (accp[0] + accp[1] + tblp[...]) * dinv[...]
    t_o[...] = jnp.dot(u, w[...], preferred_element_type=_f32) + bb[0]


def _s5(accf, tblf, dinvf, wt2big, bt2):
    return pl.pallas_call(
        _s5_body,
        grid=(FGRID,),
        in_specs=[_accspec(), _fspec(), _fspec(),
                  _wspec((128, 8)), _wspec((1,))],
        out_specs=[pl.BlockSpec((FBLK, 8), lambda i: (i, 0))],
        out_shape=[jax.ShapeDtypeStruct((NF, 8), _f32)],
    )(accf, tblf, dinvf, wt2big, bt2)[0]


def _row_body(xf, w1, b1, w2, b2, w3, b3, w4, b4, r2_o, tr_o):
    h1 = jnp.maximum(
        jnp.dot(xf[0], w1[...], preferred_element_type=_f32) + b1[...][None, :], 0.0)
    h2 = jnp.maximum(
        jnp.dot(h1, w2[...], preferred_element_type=_f32) + b2[...][None, :], 0.0)
    r2_o[0] = h2
    h3 = jnp.maximum(
        jnp.dot(h2, w3[...], preferred_element_type=_f32) + b3[...][None, :], 0.0)
    tr_o[0] = jnp.dot(h3, w4[...], preferred_element_type=_f32) + b4[0]


def _row(xrf, w1big, be1f, w2big, be2f, w3big, bt1f, wt2big, bt2):
    def bspec(minor):
        return pl.BlockSpec((1, FBLK, minor), lambda b, i: (b, i, 0))

    def wspec(shape):
        return pl.BlockSpec(shape, lambda b, i: tuple(0 for _ in shape))

    return pl.pallas_call(
        _row_body,
        grid=(3, FGRID),
        in_specs=[bspec(128),
                  wspec((128, 128)), wspec((128,)),
                  wspec((128, 128)), wspec((128,)),
                  wspec((128, 128)), wspec((128,)),
                  wspec((128, 8)), wspec((1,))],
        out_specs=[bspec(128), bspec(8)],
        out_shape=[
            jax.ShapeDtypeStruct((3, NF, 128), _f32),
            jax.ShapeDtypeStruct((3, NF, 8), _f32),
        ],
    )(xrf, w1big, be1f, w2big, be2f, w3big, bt1f, wt2big, bt2)


def _pinmlp_body(sf, recip, wp1, bp1, wp2, bp2, out_o):
    m = (sf[0] + sf[1]) * recip[...][None, :]
    g = jnp.dot(m, wp1[...], preferred_element_type=_f32) + bp1[...][None, :]
    g = jnp.maximum(g, 0.0)
    out_o[...] = jnp.dot(g, wp2[...], preferred_element_type=_f32) + bp2[...][None, :]


def _pinmlp(sf, recip, wp1, bp1, wp2, bp2):
    return pl.pallas_call(
        _pinmlp_body,
        out_shape=jax.ShapeDtypeStruct((B, PINS), _f32),
    )(sf, recip, wp1, bp1, wp2, bp2)


def _kron8(w):
    return jnp.kron(jnp.eye(8, dtype=_f32), w.astype(_f32))


def kernel(probe_temperature, probe_locations, cell_adjacency, cell_to_pin_mapping,
           We1, be1, We2, be2, Wp1, bp1, Wp2, bp2, Wt1, bt1, Wt2, bt2):
    src = cell_adjacency[0]
    dst = cell_adjacency[1]
    pvp = jnp.zeros((B, NP), _f32).at[:, probe_locations].set(probe_temperature)
    pm = jnp.zeros((NP,), _f32).at[probe_locations].set(1.0)

    epad = jnp.full((EP - E,), NP - 1, jnp.int32)
    srcw = jnp.concatenate([src, epad]).reshape(NW, NCHUNK, 128)
    dstw = jnp.concatenate([dst, epad]).reshape(NW, NCHUNK, 128)

    valid = cell_to_pin_mapping >= 0
    idxp = jnp.where(valid, cell_to_pin_mapping, TRASH).astype(jnp.int32)
    idxw = jnp.concatenate(
        [idxp, jnp.full((NP - N,), TRASH, jnp.int32)]).reshape(8, PCH, 128)
    pww = jnp.concatenate(
        [valid.astype(_f32), jnp.zeros((NP - N,), _f32)]).reshape(8, PCH, 128)

    z16 = jnp.zeros((NP, 16), _f32)
    zp16 = jnp.zeros((PP, 16), _f32)
    zp = jnp.zeros((PP,), _f32)
    ones16 = jnp.ones((128, 16), _f32)

    col = jnp.arange(128)
    csel = col // 16
    pv0r = jnp.take(pvp[0].reshape(NF, 8), csel, axis=1)
    pmr = jnp.take(pm.reshape(NF, 8), csel, axis=1)
    x0f = jnp.where((col % 16) == 0, pv0r, jnp.where((col % 16) == 1, pmr, 0.0))
    pvrr = jnp.take(pvp[1:].reshape(3, NF, 8), csel, axis=2)
    pmr3 = jnp.broadcast_to(pmr[None], (3, NF, 128))
    xrf = jnp.where((col % 16) == 0, pvrr,
                    jnp.where((col % 16) == 1, pmr3, 0.0))

    w1big = _kron8(jnp.zeros((16, 16), _f32).at[:2, :].set(We1))
    w2big = _kron8(We2)
    w3big = _kron8(Wt1)
    wt2big = _kron8(Wt2)
    be1f = jnp.tile(be1, 8)
    be2f = jnp.tile(be2, 8)
    bt1f = jnp.tile(bt1, 8)

    degp = _deg(dstw, ones16, z16)
    degf = degp.reshape(2, NF, 128)
    dinvf, tbl1f = _s1(degf, x0f)

    accf1 = _agg(tbl1f.reshape(NP, 16), srcw, dstw, z16).reshape(2, NF, 128)
    (tbl2f,) = _stage(accf1, tbl1f, dinvf, w1big, be1f)
    accf2 = _agg(tbl2f.reshape(NP, 16), srcw, dstw, z16).reshape(2, NF, 128)
    tbl3f, h2f = _stage(accf2, tbl2f, dinvf, w2big, be2f, emit_h=True)
    accf3 = _agg(tbl3f.reshape(NP, 16), srcw, dstw, z16).reshape(2, NF, 128)
    (tbl4f,) = _stage(accf3, tbl3f, dinvf, w3big, bt1f)
    accf4 = _agg(tbl4f.reshape(NP, 16), srcw, dstw, z16).reshape(2, NF, 128)
    t0f = _s5(accf4, tbl4f, dinvf, wt2big, bt2)

    r2f, trf = _row(xrf, w1big, be1f, w2big, be2f, w3big, bt1f, wt2big, bt2)

    sumsp, cntsp = _pin(h2f.reshape(NP, 16), r2f.reshape(3, NP, 16),
                        idxw, pww, zp16, zp)
    sf = sumsp.reshape(2, B, PP * 16)[:, :, :TRASH * 16]
    counts = cntsp[0] + cntsp[1]
    recip = jnp.repeat(1.0 / jnp.clip(counts[:TRASH], 1.0, None), 16)

    pin_power = _pinmlp(sf, recip, Wp1, bp1, Wp2, bp2)
    temperature = jnp.concatenate(
        [t0f.reshape(NP)[None, :N], trf.reshape(3, NP)[:, :N]], axis=0)
    return pin_power, temperature


# speedup vs baseline: 85.1947x; 1.1136x over previous
import jax
import jax.numpy as jnp
from jax import lax
from jax.experimental import pallas as pl
from jax.experimental.pallas import tpu as pltpu
from jax.experimental.pallas import tpu_sc as plsc

N = 50000
B = 4
PINS = 512
E = 800000
NP = 51200
SLC = NP // 16
PP = 640
TRASH = 512
NW = 32
NCHUNK = 196
EPT = NCHUNK * 128
EP = EPT * NW
PCH = NP // 8 // 128
NF = NP // 8
FBLK = 256
FGRID = NF // FBLK

_mesh = plsc.VectorSubcoreMesh(core_axis_name="c", subcore_axis_name="s")
_f32 = jnp.float32


def _deg_body(dstw, ones_h, z16, out, dstv, onesv, acc):
    cid = lax.axis_index("c")
    sid = lax.axis_index("s")
    wid = cid * 16 + sid
    r0 = sid * SLC
    pltpu.sync_copy(z16.at[pl.ds(r0, SLC)], acc.at[pl.ds(r0, SLC)])
    pltpu.sync_copy(dstw.at[wid], dstv)
    pltpu.sync_copy(ones_h, onesv)
    plsc.subcore_barrier()

    @pl.loop(0, NCHUNK)
    def _chunk(j):
        pltpu.sync_copy(onesv, acc.at[dstv.at[j]], add=True)

    plsc.subcore_barrier()
    pltpu.sync_copy(acc.at[pl.ds(r0, SLC)], out.at[cid].at[pl.ds(r0, SLC)])


_deg = pl.kernel(
    _deg_body,
    out_type=jax.ShapeDtypeStruct((2, NP, 16), _f32),
    mesh=_mesh,
    compiler_params=pltpu.CompilerParams(use_tc_tiling_on_sc=False),
    scratch_types=[
        pltpu.VMEM((NCHUNK, 128), jnp.int32),
        pltpu.VMEM((128, 16), _f32),
        pltpu.VMEM_SHARED((NP, 16), _f32),
    ],
)


def _agg_body(tbl, srcw, dstw, z16, out, srcv, dstv, r0b, r1b, r2b, r3b, acc,
              sg0, sg1, sg2, sg3, ss0, ss1, ss2, ss3):
    rowsb = [r0b, r1b, r2b, r3b]
    semg = [sg0, sg1, sg2, sg3]
    sems = [ss0, ss1, ss2, ss3]
    cid = lax.axis_index("c")
    sid = lax.axis_index("s")
    wid = cid * 16 + sid
    r0 = sid * SLC
    pltpu.sync_copy(z16.at[pl.ds(r0, SLC)], acc.at[pl.ds(r0, SLC)])
    pltpu.sync_copy(srcw.at[wid], srcv)
    pltpu.sync_copy(dstw.at[wid], dstv)
    plsc.subcore_barrier()

    for b in range(4):
        pltpu.async_copy(tbl.at[srcv.at[b]], rowsb[b], semg[b])

    @pl.loop(0, NCHUNK // 4)
    def _grp(k):
        for b in range(4):
            j = 4 * k + b
            pltpu.make_async_copy(tbl.at[srcv.at[j]], rowsb[b], semg[b]).wait()
            pltpu.async_copy(rowsb[b], acc.at[dstv.at[j]], sems[b], add=True)

            @pl.when(k < NCHUNK // 4 - 1)
            def _refill():
                pltpu.make_async_copy(rowsb[b], acc.at[dstv.at[j]],
                                      sems[b]).wait()
                pltpu.async_copy(tbl.at[srcv.at[j + 4]], rowsb[b], semg[b])

    for b in range(4):
        j = NCHUNK - 4 + b
        pltpu.make_async_copy(rowsb[b], acc.at[dstv.at[j]], sems[b]).wait()

    plsc.subcore_barrier()
    pltpu.sync_copy(acc.at[pl.ds(r0, SLC)], out.at[cid].at[pl.ds(r0, SLC)])


_agg = pl.kernel(
    _agg_body,
    out_type=jax.ShapeDtypeStruct((2, NP, 16), _f32),
    mesh=_mesh,
    compiler_params=pltpu.CompilerParams(use_tc_tiling_on_sc=False),
    scratch_types=[
        pltpu.VMEM((NCHUNK, 128), jnp.int32),
        pltpu.VMEM((NCHUNK, 128), jnp.int32),
        pltpu.VMEM((128, 16), _f32),
        pltpu.VMEM((128, 16), _f32),
        pltpu.VMEM((128, 16), _f32),
        pltpu.VMEM((128, 16), _f32),
        pltpu.VMEM_SHARED((NP, 16), _f32),
        pltpu.SemaphoreType.DMA,
        pltpu.SemaphoreType.DMA,
        pltpu.SemaphoreType.DMA,
        pltpu.SemaphoreType.DMA,
        pltpu.SemaphoreType.DMA,
        pltpu.SemaphoreType.DMA,
        pltpu.SemaphoreType.DMA,
        pltpu.SemaphoreType.DMA,
    ],
)


def _pin_body(h2, r2, idxw, pww, zp16, zp, sums, cnts, idxv, pwv,
              rows0, rows1, acc, cacc, sg0, sg1, ss0, ss1):
    rowsb = [rows0, rows1]
    semg = [sg0, sg1]
    sems = [ss0, ss1]
    cid = lax.axis_index("c")
    sid = lax.axis_index("s")
    b = sid % 4
    g = sid // 4
    gg = cid * 4 + g
    rowbase = gg * (NP // 8)

    @pl.when(g == 0)
    def _zero():
        pltpu.sync_copy(zp16, acc.at[b])

    @pl.when(sid == 0)
    def _zeroc():
        pltpu.sync_copy(zp, cacc)

    pltpu.sync_copy(idxw.at[gg], idxv)
    pltpu.sync_copy(pww.at[gg], pwv)
    plsc.subcore_barrier()

    def _load(j, rb, sg):
        @pl.when(b == 0)
        def _g0():
            pltpu.async_copy(h2.at[pl.ds(rowbase + j * 128, 128)], rb, sg)

        @pl.when(b != 0)
        def _gr():
            pltpu.async_copy(r2.at[b - 1].at[pl.ds(rowbase + j * 128, 128)],
                             rb, sg)

    def _wait_load(j, rb, sg):
        pltpu.make_async_copy(h2.at[pl.ds(rowbase + j * 128, 128)],
                              rb, sg).wait()

    for bb in range(2):
        _load(bb, rowsb[bb], semg[bb])

    @pl.loop(0, PCH // 2)
    def _grp(k):
        for bb in range(2):
            j = 2 * k + bb
            _wait_load(j, rowsb[bb], semg[bb])
            pltpu.async_copy(rowsb[bb], acc.at[b].at[idxv.at[j]],
                             sems[bb], add=True)

            @pl.when(b == 0)
            def _cnt():
                pltpu.sync_copy(pwv.at[j], cacc.at[idxv.at[j]], add=True)

            @pl.when(k < PCH // 2 - 1)
            def _refill():
                pltpu.make_async_copy(rowsb[bb], acc.at[b].at[idxv.at[j]],
                                      sems[bb]).wait()
                _load(j + 2, rowsb[bb], semg[bb])

    for bb in range(2):
        j = PCH - 2 + bb
        pltpu.make_async_copy(rowsb[bb], acc.at[b].at[idxv.at[j]],
                              sems[bb]).wait()

    plsc.subcore_barrier()

    @pl.when(g == 0)
    def _out():
        pltpu.sync_copy(acc.at[b], sums.at[cid].at[b])

    @pl.when(sid == 0)
    def _outc():
        pltpu.sync_copy(cacc, cnts.at[cid])


_pin = pl.kernel(
    _pin_body,
    out_type=(
        jax.ShapeDtypeStruct((2, B, PP, 16), _f32),
        jax.ShapeDtypeStruct((2, PP), _f32),
    ),
    mesh=_mesh,
    compiler_params=pltpu.CompilerParams(use_tc_tiling_on_sc=False),
    scratch_types=[
        pltpu.VMEM((PCH, 128), jnp.int32),
        pltpu.VMEM((PCH, 128), _f32),
        pltpu.VMEM((128, 16), _f32),
        pltpu.VMEM((128, 16), _f32),
        pltpu.VMEM_SHARED((B, PP, 16), _f32),
        pltpu.VMEM_SHARED((PP,), _f32),
        pltpu.SemaphoreType.DMA,
        pltpu.SemaphoreType.DMA,
        pltpu.SemaphoreType.DMA,
        pltpu.SemaphoreType.DMA,
    ],
)


def _fspec():
    return pl.BlockSpec((FBLK, 128), lambda i: (i, 0))


def _wspec(shape):
    return pl.BlockSpec(shape, lambda i: tuple(0 for _ in shape))


def _s1_body(degf, x0f, dinv_o, tbl_o):
    dv = lax.rsqrt(degf[0] + degf[1] + 1.0)
    dinv_o[...] = dv
    tbl_o[...] = dv * x0f[...]


def _s1(degf, x0f):
    return pl.pallas_call(
        _s1_body,
        grid=(FGRID,),
        in_specs=[_accspec(), _fspec()],
        out_specs=[_fspec(), _fspec()],
        out_shape=[
            jax.ShapeDtypeStruct((NF, 128), _f32),
            jax.ShapeDtypeStruct((NF, 128), _f32),
        ],
    )(degf, x0f)


def _accspec():
    return pl.BlockSpec((2, FBLK, 128), lambda i: (0, i, 0))


def _make_stage(emit_h):
    def body(accp, tblp, dinv, w, bb, *outs):
        dv = dinv[...]
        u = (accp[0] + accp[1] + tblp[...]) * dv
        h = jnp.dot(u, w[...], preferred_element_type=_f32) + bb[...][None, :]
        h = jnp.maximum(h, 0.0)
        outs[0][...] = h * dv
        if emit_h:
            outs[1][...] = h
    return body


def _stage(accf, tblf, dinvf, wbig, bflat, emit_h=False):
    n_out = 2 if emit_h else 1
    out_shape = [jax.ShapeDtypeStruct((NF, 128), _f32)] * n_out
    return pl.pallas_call(
        _make_stage(emit_h),
        grid=(FGRID,),
        in_specs=[_accspec(), _fspec(), _fspec(),
                  _wspec((128, 128)), _wspec((128,))],
        out_specs=[_fspec()] * n_out,
        out_shape=out_shape,
    )(accf, tblf, dinvf, wbig, bflat)


def _s5_body(accp, tblp, dinv, w, bb, t_o):
    u = (accp[0] + accp[1] + tblp[...]) * dinv[...]
    t_o[...] = jnp.dot(u, w[...], preferred_element_type=_f32) + bb[0]


def _s5(accf, tblf, dinvf, wt2big, bt2):
    return pl.pallas_call(
        _s5_body,
        grid=(FGRID,),
        in_specs=[_accspec(), _fspec(), _fspec(),
                  _wspec((128, 8)), _wspec((1,))],
        out_specs=[pl.BlockSpec((FBLK, 8), lambda i: (i, 0))],
        out_shape=[jax.ShapeDtypeStruct((NF, 8), _f32)],
    )(accf, tblf, dinvf, wt2big, bt2)[0]


def _row_body(xf, w1, b1, w2, b2, w3, b3, w4, b4, r2_o, tr_o):
    h1 = jnp.maximum(
        jnp.dot(xf[0], w1[...], preferred_element_type=_f32) + b1[...][None, :], 0.0)
    h2 = jnp.maximum(
        jnp.dot(h1, w2[...], preferred_element_type=_f32) + b2[...][None, :], 0.0)
    r2_o[0] = h2
    h3 = jnp.maximum(
        jnp.dot(h2, w3[...], preferred_element_type=_f32) + b3[...][None, :], 0.0)
    tr_o[0] = jnp.dot(h3, w4[...], preferred_element_type=_f32) + b4[0]


def _row(xrf, w1big, be1f, w2big, be2f, w3big, bt1f, wt2big, bt2):
    def bspec(minor):
        return pl.BlockSpec((1, FBLK, minor), lambda b, i: (b, i, 0))

    def wspec(shape):
        return pl.BlockSpec(shape, lambda b, i: tuple(0 for _ in shape))

    return pl.pallas_call(
        _row_body,
        grid=(3, FGRID),
        in_specs=[bspec(128),
                  wspec((128, 128)), wspec((128,)),
                  wspec((128, 128)), wspec((128,)),
                  wspec((128, 128)), wspec((128,)),
                  wspec((128, 8)), wspec((1,))],
        out_specs=[bspec(128), bspec(8)],
        out_shape=[
            jax.ShapeDtypeStruct((3, NF, 128), _f32),
            jax.ShapeDtypeStruct((3, NF, 8), _f32),
        ],
    )(xrf, w1big, be1f, w2big, be2f, w3big, bt1f, wt2big, bt2)


def _pinmlp_body(sf, recip, wp1, bp1, wp2, bp2, out_o):
    m = (sf[0] + sf[1]) * recip[...][None, :]
    g = jnp.dot(m, wp1[...], preferred_element_type=_f32) + bp1[...][None, :]
    g = jnp.maximum(g, 0.0)
    out_o[...] = jnp.dot(g, wp2[...], preferred_element_type=_f32) + bp2[...][None, :]


def _pinmlp(sf, recip, wp1, bp1, wp2, bp2):
    return pl.pallas_call(
        _pinmlp_body,
        out_shape=jax.ShapeDtypeStruct((B, PINS), _f32),
    )(sf, recip, wp1, bp1, wp2, bp2)


def _kron8(w):
    return jnp.kron(jnp.eye(8, dtype=_f32), w.astype(_f32))


def kernel(probe_temperature, probe_locations, cell_adjacency, cell_to_pin_mapping,
           We1, be1, We2, be2, Wp1, bp1, Wp2, bp2, Wt1, bt1, Wt2, bt2):
    src = cell_adjacency[0]
    dst = cell_adjacency[1]
    pvp = jnp.zeros((B, NP), _f32).at[:, probe_locations].set(probe_temperature)
    pm = jnp.zeros((NP,), _f32).at[probe_locations].set(1.0)

    epad = jnp.full((EP - E,), NP - 1, jnp.int32)
    srcw = jnp.concatenate([src, epad]).reshape(NW, NCHUNK, 128)
    dstw = jnp.concatenate([dst, epad]).reshape(NW, NCHUNK, 128)

    valid = cell_to_pin_mapping >= 0
    idxp = jnp.where(valid, cell_to_pin_mapping, TRASH).astype(jnp.int32)
    idxw = jnp.concatenate(
        [idxp, jnp.full((NP - N,), TRASH, jnp.int32)]).reshape(8, PCH, 128)
    pww = jnp.concatenate(
        [valid.astype(_f32), jnp.zeros((NP - N,), _f32)]).reshape(8, PCH, 128)

    z16 = jnp.zeros((NP, 16), _f32)
    zp16 = jnp.zeros((PP, 16), _f32)
    zp = jnp.zeros((PP,), _f32)
    ones16 = jnp.ones((128, 16), _f32)

    col = jnp.arange(128)
    csel = col // 16
    pv0r = jnp.take(pvp[0].reshape(NF, 8), csel, axis=1)
    pmr = jnp.take(pm.reshape(NF, 8), csel, axis=1)
    x0f = jnp.where((col % 16) == 0, pv0r, jnp.where((col % 16) == 1, pmr, 0.0))
    pvrr = jnp.take(pvp[1:].reshape(3, NF, 8), csel, axis=2)
    pmr3 = jnp.broadcast_to(pmr[None], (3, NF, 128))
    xrf = jnp.where((col % 16) == 0, pvrr,
                    jnp.where((col % 16) == 1, pmr3, 0.0))

    w1big = _kron8(jnp.zeros((16, 16), _f32).at[:2, :].set(We1))
    w2big = _kron8(We2)
    w3big = _kron8(Wt1)
    wt2big = _kron8(Wt2)
    be1f = jnp.tile(be1, 8)
    be2f = jnp.tile(be2, 8)
    bt1f = jnp.tile(bt1, 8)

    degp = _deg(dstw, ones16, z16)
    degf = degp.reshape(2, NF, 128)
    dinvf, tbl1f = _s1(degf, x0f)

    accf1 = _agg(tbl1f.reshape(NP, 16), srcw, dstw, z16).reshape(2, NF, 128)
    (tbl2f,) = _stage(accf1, tbl1f, dinvf, w1big, be1f)
    accf2 = _agg(tbl2f.reshape(NP, 16), srcw, dstw, z16).reshape(2, NF, 128)
    tbl3f, h2f = _stage(accf2, tbl2f, dinvf, w2big, be2f, emit_h=True)
    accf3 = _agg(tbl3f.reshape(NP, 16), srcw, dstw, z16).reshape(2, NF, 128)
    (tbl4f,) = _stage(accf3, tbl3f, dinvf, w3big, bt1f)
    accf4 = _agg(tbl4f.reshape(NP, 16), srcw, dstw, z16).reshape(2, NF, 128)
    t0f = _s5(accf4, tbl4f, dinvf, wt2big, bt2)

    r2f, trf = _row(xrf, w1big, be1f, w2big, be2f, w3big, bt1f, wt2big, bt2)

    sumsp, cntsp = _pin(h2f.reshape(NP, 16), r2f.reshape(3, NP, 16),
                        idxw, pww, zp16, zp)
    sf = sumsp.reshape(2, B, PP * 16)[:, :, :TRASH * 16]
    counts = cntsp[0] + cntsp[1]
    recip = jnp.repeat(1.0 / jnp.clip(counts[:TRASH], 1.0, None), 16)

    pin_power = _pinmlp(sf, recip, Wp1, bp1, Wp2, bp2)
    temperature = jnp.concatenate(
        [t0f.reshape(NP)[None, :N], trf.reshape(3, NP)[:, :N]], axis=0)
    return pin_power, temperature


# speedup vs baseline: 95.6205x; 1.1224x over previous
import jax
import jax.numpy as jnp
from jax import lax
from jax.experimental import pallas as pl
from jax.experimental.pallas import tpu as pltpu
from jax.experimental.pallas import tpu_sc as plsc

N = 50000
B = 4
PINS = 512
E = 800000
NP = 51200
SLC = NP // 16
PP = 640
TRASH = 512
NW = 32
NCHUNK = 200
EPT = NCHUNK * 128
EP = EPT * NW
PCH = NP // 8 // 128
NF = NP // 8
FBLK = 256
FGRID = NF // FBLK

_mesh = plsc.VectorSubcoreMesh(core_axis_name="c", subcore_axis_name="s")
_f32 = jnp.float32


def _deg_body(dstw, ones_h, z16, out, dstv, onesv, acc):
    cid = lax.axis_index("c")
    sid = lax.axis_index("s")
    wid = cid * 16 + sid
    r0 = sid * SLC
    pltpu.sync_copy(z16.at[pl.ds(r0, SLC)], acc.at[pl.ds(r0, SLC)])
    pltpu.sync_copy(dstw.at[wid], dstv)
    pltpu.sync_copy(ones_h, onesv)
    plsc.subcore_barrier()

    @pl.loop(0, NCHUNK)
    def _chunk(j):
        pltpu.sync_copy(onesv, acc.at[dstv.at[j]], add=True)

    plsc.subcore_barrier()
    pltpu.sync_copy(acc.at[pl.ds(r0, SLC)], out.at[cid].at[pl.ds(r0, SLC)])


_deg = pl.kernel(
    _deg_body,
    out_type=jax.ShapeDtypeStruct((2, NP, 16), _f32),
    mesh=_mesh,
    compiler_params=pltpu.CompilerParams(use_tc_tiling_on_sc=False),
    scratch_types=[
        pltpu.VMEM((NCHUNK, 128), jnp.int32),
        pltpu.VMEM((128, 16), _f32),
        pltpu.VMEM_SHARED((NP, 16), _f32),
    ],
)


def _agg_body(tbl, srcw, dstw, z16, out, srcv, dstv,
              rb0, rb1, rb2, rb3, rb4, rb5, rb6, rb7, acc,
              sg0, sg1, sg2, sg3, sg4, sg5, sg6, sg7,
              ss0, ss1, ss2, ss3, ss4, ss5, ss6, ss7):
    rowsb = [rb0, rb1, rb2, rb3, rb4, rb5, rb6, rb7]
    semg = [sg0, sg1, sg2, sg3, sg4, sg5, sg6, sg7]
    sems = [ss0, ss1, ss2, ss3, ss4, ss5, ss6, ss7]
    cid = lax.axis_index("c")
    sid = lax.axis_index("s")
    wid = cid * 16 + sid
    r0 = sid * SLC
    pltpu.sync_copy(z16.at[pl.ds(r0, SLC)], acc.at[pl.ds(r0, SLC)])
    pltpu.sync_copy(srcw.at[wid], srcv)
    pltpu.sync_copy(dstw.at[wid], dstv)
    plsc.subcore_barrier()

    for b in range(4):
        pltpu.async_copy(tbl.at[srcv.at[b]], rowsb[b], semg[b])

    @pl.loop(0, NCHUNK // 8)
    def _grp(k):
        for b in range(8):
            j = 8 * k + b
            b2 = (b + 4) % 8
            pltpu.make_async_copy(tbl.at[srcv.at[j]], rowsb[b], semg[b]).wait()
            pltpu.async_copy(rowsb[b], acc.at[dstv.at[j]], sems[b], add=True)
            if b < 4:
                @pl.when(k > 0)
                def _retire():
                    pltpu.make_async_copy(rowsb[b2], acc.at[dstv.at[j - 4]],
                                          sems[b2]).wait()
                pltpu.async_copy(tbl.at[srcv.at[j + 4]], rowsb[b2], semg[b2])
            else:
                pltpu.make_async_copy(rowsb[b2], acc.at[dstv.at[j - 4]],
                                      sems[b2]).wait()

                @pl.when(k < NCHUNK // 8 - 1)
                def _refill():
                    pltpu.async_copy(tbl.at[srcv.at[j + 4]], rowsb[b2],
                                     semg[b2])

    for b in range(4, 8):
        j = NCHUNK - 8 + b
        pltpu.make_async_copy(rowsb[b], acc.at[dstv.at[j]], sems[b]).wait()

    plsc.subcore_barrier()
    pltpu.sync_copy(acc.at[pl.ds(r0, SLC)], out.at[cid].at[pl.ds(r0, SLC)])


_agg = pl.kernel(
    _agg_body,
    out_type=jax.ShapeDtypeStruct((2, NP, 16), _f32),
    mesh=_mesh,
    compiler_params=pltpu.CompilerParams(use_tc_tiling_on_sc=False),
    scratch_types=(
        [pltpu.VMEM((NCHUNK, 128), jnp.int32)] * 2
        + [pltpu.VMEM((128, 16), _f32)] * 8
        + [pltpu.VMEM_SHARED((NP, 16), _f32)]
        + [pltpu.SemaphoreType.DMA] * 16
    ),
)


def _pin_body(h2, r2, idxw, pww, zp16, zp, sums, cnts, idxv, pwv,
              rows0, rows1, acc, cacc, sg0, sg1, ss0, ss1):
    rowsb = [rows0, rows1]
    semg = [sg0, sg1]
    sems = [ss0, ss1]
    cid = lax.axis_index("c")
    sid = lax.axis_index("s")
    b = sid % 4
    g = sid // 4
    gg = cid * 4 + g
    rowbase = gg * (NP // 8)

    @pl.when(g == 0)
    def _zero():
        pltpu.sync_copy(zp16, acc.at[b])

    @pl.when(sid == 0)
    def _zeroc():
        pltpu.sync_copy(zp, cacc)

    pltpu.sync_copy(idxw.at[gg], idxv)
    pltpu.sync_copy(pww.at[gg], pwv)
    plsc.subcore_barrier()

    def _load(j, rb, sg):
        @pl.when(b == 0)
        def _g0():
            pltpu.async_copy(h2.at[pl.ds(rowbase + j * 128, 128)], rb, sg)

        @pl.when(b != 0)
        def _gr():
            pltpu.async_copy(r2.at[b - 1].at[pl.ds(rowbase + j * 128, 128)],
                             rb, sg)

    def _wait_load(j, rb, sg):
        pltpu.make_async_copy(h2.at[pl.ds(rowbase + j * 128, 128)],
                              rb, sg).wait()

    for bb in range(2):
        _load(bb, rowsb[bb], semg[bb])

    @pl.loop(0, PCH // 2)
    def _grp(k):
        for bb in range(2):
            j = 2 * k + bb
            _wait_load(j, rowsb[bb], semg[bb])
            pltpu.async_copy(rowsb[bb], acc.at[b].at[idxv.at[j]],
                             sems[bb], add=True)

            @pl.when(b == 0)
            def _cnt():
                pltpu.sync_copy(pwv.at[j], cacc.at[idxv.at[j]], add=True)

            @pl.when(k < PCH // 2 - 1)
            def _refill():
                pltpu.make_async_copy(rowsb[bb], acc.at[b].at[idxv.at[j]],
                                      sems[bb]).wait()
                _load(j + 2, rowsb[bb], semg[bb])

    for bb in range(2):
        j = PCH - 2 + bb
        pltpu.make_async_copy(rowsb[bb], acc.at[b].at[idxv.at[j]],
                              sems[bb]).wait()

    plsc.subcore_barrier()

    @pl.when(g == 0)
    def _out():
        pltpu.sync_copy(acc.at[b], sums.at[cid].at[b])

    @pl.when(sid == 0)
    def _outc():
        pltpu.sync_copy(cacc, cnts.at[cid])


_pin = pl.kernel(
    _pin_body,
    out_type=(
        jax.ShapeDtypeStruct((2, B, PP, 16), _f32),
        jax.ShapeDtypeStruct((2, PP), _f32),
    ),
    mesh=_mesh,
    compiler_params=pltpu.CompilerParams(use_tc_tiling_on_sc=False),
    scratch_types=[
        pltpu.VMEM((PCH, 128), jnp.int32),
        pltpu.VMEM((PCH, 128), _f32),
        pltpu.VMEM((128, 16), _f32),
        pltpu.VMEM((128, 16), _f32),
        pltpu.VMEM_SHARED((B, PP, 16), _f32),
        pltpu.VMEM_SHARED((PP,), _f32),
        pltpu.SemaphoreType.DMA,
        pltpu.SemaphoreType.DMA,
        pltpu.SemaphoreType.DMA,
        pltpu.SemaphoreType.DMA,
    ],
)


def _fspec():
    return pl.BlockSpec((FBLK, 128), lambda i: (i, 0))


def _wspec(shape):
    return pl.BlockSpec(shape, lambda i: tuple(0 for _ in shape))


def _s1_body(degf, x0f, dinv_o, tbl_o):
    dv = lax.rsqrt(degf[0] + degf[1] + 1.0)
    dinv_o[...] = dv
    tbl_o[...] = dv * x0f[...]


def _s1(degf, x0f):
    return pl.pallas_call(
        _s1_body,
        grid=(FGRID,),
        in_specs=[_accspec(), _fspec()],
        out_specs=[_fspec(), _fspec()],
        out_shape=[
            jax.ShapeDtypeStruct((NF, 128), _f32),
            jax.ShapeDtypeStruct((NF, 128), _f32),
        ],
    )(degf, x0f)


def _accspec():
    return pl.BlockSpec((2, FBLK, 128), lambda i: (0, i, 0))


def _make_stage(emit_h):
    def body(accp, tblp, dinv, w, bb, *outs):
        dv = dinv[...]
        u = (accp[0] + accp[1] + tblp[...]) * dv
        h = jnp.dot(u, w[...], preferred_element_type=_f32) + bb[...][None, :]
        h = jnp.maximum(h, 0.0)
        outs[0][...] = h * dv
        if emit_h:
            outs[1][...] = h
    return body


def _stage(accf, tblf, dinvf, wbig, bflat, emit_h=False):
    n_out = 2 if emit_h else 1
    out_shape = [jax.ShapeDtypeStruct((NF, 128), _f32)] * n_out
    return pl.pallas_call(
        _make_stage(emit_h),
        grid=(FGRID,),
        in_specs=[_accspec(), _fspec(), _fspec(),
                  _wspec((128, 128)), _wspec((128,))],
        out_specs=[_fspec()] * n_out,
        out_shape=out_shape,
    )(accf, tblf, dinvf, wbig, bflat)


def _s5_body(accp, tblp, dinv, w, bb, t_o):
    u = (accp[0] + accp[1] + tblp[...]) * dinv[...]
    t_o[...] = jnp.dot(u, w[...], preferred_element_type=_f32) + bb[0]


def _s5(accf, tblf, dinvf, wt2big, bt2):
    return pl.pallas_call(
        _s5_body,
        grid=(FGRID,),
        in_specs=[_accspec(), _fspec(), _fspec(),
                  _wspec((128, 8)), _wspec((1,))],
        out_specs=[pl.BlockSpec((FBLK, 8), lambda i: (i, 0))],
        out_shape=[jax.ShapeDtypeStruct((NF, 8), _f32)],
    )(accf, tblf, dinvf, wt2big, bt2)[0]


def _row_body(xf, w1, b1, w2, b2, w3, b3, w4, b4, r2_o, tr_o):
    h1 = jnp.maximum(
        jnp.dot(xf[0], w1[...], preferred_element_type=_f32) + b1[...][None, :], 0.0)
    h2 = jnp.maximum(
        jnp.dot(h1, w2[...], preferred_element_type=_f32) + b2[...][None, :], 0.0)
    r2_o[0] = h2
    h3 = jnp.maximum(
        jnp.dot(h2, w3[...], preferred_element_type=_f32) + b3[...][None, :], 0.0)
    tr_o[0] = jnp.dot(h3, w4[...], preferred_element_type=_f32) + b4[0]


def _row(xrf, w1big, be1f, w2big, be2f, w3big, bt1f, wt2big, bt2):
    def bspec(minor):
        return pl.BlockSpec((1, FBLK, minor), lambda b, i: (b, i, 0))

    def wspec(shape):
        return pl.BlockSpec(shape, lambda b, i: tuple(0 for _ in shape))

    return pl.pallas_call(
        _row_body,
        grid=(3, FGRID),
        in_specs=[bspec(128),
                  wspec((128, 128)), wspec((128,)),
                  wspec((128, 128)), wspec((128,)),
                  wspec((128, 128)), wspec((128,)),
                  wspec((128, 8)), wspec((1,))],
        out_specs=[bspec(128), bspec(8)],
        out_shape=[
            jax.ShapeDtypeStruct((3, NF, 128), _f32),
            jax.ShapeDtypeStruct((3, NF, 8), _f32),
        ],
    )(xrf, w1big, be1f, w2big, be2f, w3big, bt1f, wt2big, bt2)


def _pinmlp_body(sf, recip, wp1, bp1, wp2, bp2, out_o):
    m = (sf[0] + sf[1]) * recip[...][None, :]
    g = jnp.dot(m, wp1[...], preferred_element_type=_f32) + bp1[...][None, :]
    g = jnp.maximum(g, 0.0)
    out_o[...] = jnp.dot(g, wp2[...], preferred_element_type=_f32) + bp2[...][None, :]


def _pinmlp(sf, recip, wp1, bp1, wp2, bp2):
    return pl.pallas_call(
        _pinmlp_body,
        out_shape=jax.ShapeDtypeStruct((B, PINS), _f32),
    )(sf, recip, wp1, bp1, wp2, bp2)


def _kron8(w):
    return jnp.kron(jnp.eye(8, dtype=_f32), w.astype(_f32))


def kernel(probe_temperature, probe_locations, cell_adjacency, cell_to_pin_mapping,
           We1, be1, We2, be2, Wp1, bp1, Wp2, bp2, Wt1, bt1, Wt2, bt2):
    src = cell_adjacency[0]
    dst = cell_adjacency[1]
    pvp = jnp.zeros((B, NP), _f32).at[:, probe_locations].set(probe_temperature)
    pm = jnp.zeros((NP,), _f32).at[probe_locations].set(1.0)

    epad = (N + jnp.arange(EP - E, dtype=jnp.int32) % (NP - N)).astype(jnp.int32)
    srcw = jnp.concatenate([src, epad]).reshape(NW, NCHUNK, 128)
    dstw = jnp.concatenate([dst, epad]).reshape(NW, NCHUNK, 128)

    valid = cell_to_pin_mapping >= 0
    idxp = jnp.where(valid, cell_to_pin_mapping, TRASH).astype(jnp.int32)
    idxw = jnp.concatenate(
        [idxp, jnp.full((NP - N,), TRASH, jnp.int32)]).reshape(8, PCH, 128)
    pww = jnp.concatenate(
        [valid.astype(_f32), jnp.zeros((NP - N,), _f32)]).reshape(8, PCH, 128)

    z16 = jnp.zeros((NP, 16), _f32)
    zp16 = jnp.zeros((PP, 16), _f32)
    zp = jnp.zeros((PP,), _f32)
    ones16 = jnp.ones((128, 16), _f32)

    col = jnp.arange(128)
    csel = col // 16
    pv0r = jnp.take(pvp[0].reshape(NF, 8), csel, axis=1)
    pmr = jnp.take(pm.reshape(NF, 8), csel, axis=1)
    x0f = jnp.where((col % 16) == 0, pv0r, jnp.where((col % 16) == 1, pmr, 0.0))
    pvrr = jnp.take(pvp[1:].reshape(3, NF, 8), csel, axis=2)
    pmr3 = jnp.broadcast_to(pmr[None], (3, NF, 128))
    xrf = jnp.where((col % 16) == 0, pvrr,
                    jnp.where((col % 16) == 1, pmr3, 0.0))

    w1big = _kron8(jnp.zeros((16, 16), _f32).at[:2, :].set(We1))
    w2big = _kron8(We2)
    w3big = _kron8(Wt1)
    wt2big = _kron8(Wt2)
    be1f = jnp.tile(be1, 8)
    be2f = jnp.tile(be2, 8)
    bt1f = jnp.tile(bt1, 8)

    degp = _deg(dstw, ones16, z16)
    degf = degp.reshape(2, NF, 128)
    dinvf, tbl1f = _s1(degf, x0f)

    accf1 = _agg(tbl1f.reshape(NP, 16), srcw, dstw, z16).reshape(2, NF, 128)
    (tbl2f,) = _stage(accf1, tbl1f, dinvf, w1big, be1f)
    accf2 = _agg(tbl2f.reshape(NP, 16), srcw, dstw, z16).reshape(2, NF, 128)
    tbl3f, h2f = _stage(accf2, tbl2f, dinvf, w2big, be2f, emit_h=True)
    accf3 = _agg(tbl3f.reshape(NP, 16), srcw, dstw, z16).reshape(2, NF, 128)
    (tbl4f,) = _stage(accf3, tbl3f, dinvf, w3big, bt1f)
    accf4 = _agg(tbl4f.reshape(NP, 16), srcw, dstw, z16).reshape(2, NF, 128)
    t0f = _s5(accf4, tbl4f, dinvf, wt2big, bt2)

    r2f, trf = _row(xrf, w1big, be1f, w2big, be2f, w3big, bt1f, wt2big, bt2)

    sumsp, cntsp = _pin(h2f.reshape(NP, 16), r2f.reshape(3, NP, 16),
                        idxw, pww, zp16, zp)
    sf = sumsp.reshape(2, B, PP * 16)[:, :, :TRASH * 16]
    counts = cntsp[0] + cntsp[1]
    recip = jnp.repeat(1.0 / jnp.clip(counts[:TRASH], 1.0, None), 16)

    pin_power = _pinmlp(sf, recip, Wp1, bp1, Wp2, bp2)
    temperature = jnp.concatenate(
        [t0f.reshape(NP)[None, :N], trf.reshape(3, NP)[:, :N]], axis=0)
    return pin_power, temperature


# speedup vs baseline: 105.3365x; 1.1016x over previous
import jax
import jax.numpy as jnp
from jax import lax
from jax.experimental import pallas as pl
from jax.experimental.pallas import tpu as pltpu
from jax.experimental.pallas import tpu_sc as plsc

N = 50000
B = 4
PINS = 512
E = 800000
NP = 51200
SLC = NP // 16
PP = 640
TRASH = 512
NW = 32
NCHUNK = 200
EPT = NCHUNK * 128
EP = EPT * NW
PCH = NP // 8 // 128
NF = NP // 8
FBLK = 256
FGRID = NF // FBLK

_mesh = plsc.VectorSubcoreMesh(core_axis_name="c", subcore_axis_name="s")
_f32 = jnp.float32


def _deg_body(edges, ones_h, z16, out, dstv, onesv, acc):
    cid = lax.axis_index("c")
    sid = lax.axis_index("s")
    wid = cid * 16 + sid
    r0 = sid * SLC
    pltpu.sync_copy(z16.at[pl.ds(r0, SLC)], acc.at[pl.ds(r0, SLC)])
    pltpu.sync_copy(edges.at[1].at[wid], dstv)
    pltpu.sync_copy(ones_h, onesv)
    plsc.subcore_barrier()

    @pl.loop(0, NCHUNK)
    def _chunk(j):
        pltpu.sync_copy(onesv, acc.at[dstv.at[j]], add=True)

    plsc.subcore_barrier()
    pltpu.sync_copy(acc.at[pl.ds(r0, SLC)], out.at[cid].at[pl.ds(r0, SLC)])


_deg = pl.kernel(
    _deg_body,
    out_type=jax.ShapeDtypeStruct((2, NP, 16), _f32),
    mesh=_mesh,
    compiler_params=pltpu.CompilerParams(use_tc_tiling_on_sc=False),
    scratch_types=[
        pltpu.VMEM((NCHUNK, 128), jnp.int32),
        pltpu.VMEM((128, 16), _f32),
        pltpu.VMEM_SHARED((NP, 16), _f32),
    ],
)


def _agg_body(tbl, edges, z16, out, srcv, dstv,
              rb0, rb1, rb2, rb3, rb4, rb5, rb6, rb7, acc,
              sg0, sg1, sg2, sg3, sg4, sg5, sg6, sg7,
              ss0, ss1, ss2, ss3, ss4, ss5, ss6, ss7):
    rowsb = [rb0, rb1, rb2, rb3, rb4, rb5, rb6, rb7]
    semg = [sg0, sg1, sg2, sg3, sg4, sg5, sg6, sg7]
    sems = [ss0, ss1, ss2, ss3, ss4, ss5, ss6, ss7]
    cid = lax.axis_index("c")
    sid = lax.axis_index("s")
    wid = cid * 16 + sid
    r0 = sid * SLC
    pltpu.sync_copy(z16.at[pl.ds(r0, SLC)], acc.at[pl.ds(r0, SLC)])
    pltpu.sync_copy(edges.at[0].at[wid], srcv)
    pltpu.sync_copy(edges.at[1].at[wid], dstv)
    plsc.subcore_barrier()

    for b in range(4):
        pltpu.async_copy(tbl.at[srcv.at[b]], rowsb[b], semg[b])

    @pl.loop(0, NCHUNK // 8)
    def _grp(k):
        for b in range(8):
            j = 8 * k + b
            b2 = (b + 4) % 8
            pltpu.make_async_copy(tbl.at[srcv.at[j]], rowsb[b], semg[b]).wait()
            pltpu.async_copy(rowsb[b], acc.at[dstv.at[j]], sems[b], add=True)
            if b < 4:
                @pl.when(k > 0)
                def _retire():
                    pltpu.make_async_copy(rowsb[b2], acc.at[dstv.at[j - 4]],
                                          sems[b2]).wait()
                pltpu.async_copy(tbl.at[srcv.at[j + 4]], rowsb[b2], semg[b2])
            else:
                pltpu.make_async_copy(rowsb[b2], acc.at[dstv.at[j - 4]],
                                      sems[b2]).wait()

                @pl.when(k < NCHUNK // 8 - 1)
                def _refill():
                    pltpu.async_copy(tbl.at[srcv.at[j + 4]], rowsb[b2],
                                     semg[b2])

    for b in range(4, 8):
        j = NCHUNK - 8 + b
        pltpu.make_async_copy(rowsb[b], acc.at[dstv.at[j]], sems[b]).wait()

    plsc.subcore_barrier()
    pltpu.sync_copy(acc.at[pl.ds(r0, SLC)], out.at[cid].at[pl.ds(r0, SLC)])


_agg = pl.kernel(
    _agg_body,
    out_type=jax.ShapeDtypeStruct((2, NP, 16), _f32),
    mesh=_mesh,
    compiler_params=pltpu.CompilerParams(use_tc_tiling_on_sc=False),
    scratch_types=(
        [pltpu.VMEM((NCHUNK, 128), jnp.int32)] * 2
        + [pltpu.VMEM((128, 16), _f32)] * 8
        + [pltpu.VMEM_SHARED((NP, 16), _f32)]
        + [pltpu.SemaphoreType.DMA] * 16
    ),
)


def _pin_body(h2, r2, idxw, pww, zp16, zp, sums, cnts, idxv, pwv,
              rows0, rows1, acc, cacc, sg0, sg1, ss0, ss1):
    rowsb = [rows0, rows1]
    semg = [sg0, sg1]
    sems = [ss0, ss1]
    cid = lax.axis_index("c")
    sid = lax.axis_index("s")
    b = sid % 4
    g = sid // 4
    gg = cid * 4 + g
    rowbase = gg * (NP // 8)

    @pl.when(g == 0)
    def _zero():
        pltpu.sync_copy(zp16, acc.at[b])

    @pl.when(sid == 0)
    def _zeroc():
        pltpu.sync_copy(zp, cacc)

    pltpu.sync_copy(idxw.at[gg], idxv)
    pltpu.sync_copy(pww.at[gg], pwv)
    plsc.subcore_barrier()

    def _load(j, rb, sg):
        @pl.when(b == 0)
        def _g0():
            pltpu.async_copy(h2.at[pl.ds(rowbase + j * 128, 128)], rb, sg)

        @pl.when(b != 0)
        def _gr():
            pltpu.async_copy(r2.at[b - 1].at[pl.ds(rowbase + j * 128, 128)],
                             rb, sg)

    def _wait_load(j, rb, sg):
        pltpu.make_async_copy(h2.at[pl.ds(rowbase + j * 128, 128)],
                              rb, sg).wait()

    for bb in range(2):
        _load(bb, rowsb[bb], semg[bb])

    @pl.loop(0, PCH // 2)
    def _grp(k):
        for bb in range(2):
            j = 2 * k + bb
            _wait_load(j, rowsb[bb], semg[bb])
            pltpu.async_copy(rowsb[bb], acc.at[b].at[idxv.at[j]],
                             sems[bb], add=True)

            @pl.when(b == 0)
            def _cnt():
                pltpu.sync_copy(pwv.at[j], cacc.at[idxv.at[j]], add=True)

            @pl.when(k < PCH // 2 - 1)
            def _refill():
                pltpu.make_async_copy(rowsb[bb], acc.at[b].at[idxv.at[j]],
                                      sems[bb]).wait()
                _load(j + 2, rowsb[bb], semg[bb])

    for bb in range(2):
        j = PCH - 2 + bb
        pltpu.make_async_copy(rowsb[bb], acc.at[b].at[idxv.at[j]],
                              sems[bb]).wait()

    plsc.subcore_barrier()

    @pl.when(g == 0)
    def _out():
        pltpu.sync_copy(acc.at[b], sums.at[cid].at[b])

    @pl.when(sid == 0)
    def _outc():
        pltpu.sync_copy(cacc, cnts.at[cid])


_pin = pl.kernel(
    _pin_body,
    out_type=(
        jax.ShapeDtypeStruct((2, B, PP, 16), _f32),
        jax.ShapeDtypeStruct((2, PP), _f32),
    ),
    mesh=_mesh,
    compiler_params=pltpu.CompilerParams(use_tc_tiling_on_sc=False),
    scratch_types=[
        pltpu.VMEM((PCH, 128), jnp.int32),
        pltpu.VMEM((PCH, 128), _f32),
        pltpu.VMEM((128, 16), _f32),
        pltpu.VMEM((128, 16), _f32),
        pltpu.VMEM_SHARED((B, PP, 16), _f32),
        pltpu.VMEM_SHARED((PP,), _f32),
        pltpu.SemaphoreType.DMA,
        pltpu.SemaphoreType.DMA,
        pltpu.SemaphoreType.DMA,
        pltpu.SemaphoreType.DMA,
    ],
)


def _fspec():
    return pl.BlockSpec((FBLK, 128), lambda i: (i, 0))


def _wspec(shape):
    return pl.BlockSpec(shape, lambda i: tuple(0 for _ in shape))


def _s1_body(degf, x0f, dinv_o, tbl_o):
    dv = lax.rsqrt(degf[0] + degf[1] + 1.0)
    dinv_o[...] = dv
    tbl_o[...] = dv * x0f[...]


def _s1(degf, x0f):
    return pl.pallas_call(
        _s1_body,
        grid=(FGRID,),
        in_specs=[_accspec(), _fspec()],
        out_specs=[_fspec(), _fspec()],
        out_shape=[
            jax.ShapeDtypeStruct((NF, 128), _f32),
            jax.ShapeDtypeStruct((NF, 128), _f32),
        ],
    )(degf, x0f)


def _accspec():
    return pl.BlockSpec((2, FBLK, 128), lambda i: (0, i, 0))


def _make_stage(emit_h):
    def body(accp, tblp, dinv, w, bb, *outs):
        dv = dinv[...]
        u = (accp[0] + accp[1] + tblp[...]) * dv
        h = jnp.dot(u, w[...], preferred_element_type=_f32) + bb[...][None, :]
        h = jnp.maximum(h, 0.0)
        outs[0][...] = h * dv
        if emit_h:
            outs[1][...] = h
    return body


def _stage(accf, tblf, dinvf, wbig, bflat, emit_h=False):
    n_out = 2 if emit_h else 1
    out_shape = [jax.ShapeDtypeStruct((NF, 128), _f32)] * n_out
    return pl.pallas_call(
        _make_stage(emit_h),
        grid=(FGRID,),
        in_specs=[_accspec(), _fspec(), _fspec(),
                  _wspec((128, 128)), _wspec((128,))],
        out_specs=[_fspec()] * n_out,
        out_shape=out_shape,
    )(accf, tblf, dinvf, wbig, bflat)


def _s5_body(accp, tblp, dinv, w, bb, t_o):
    u = (accp[0] + accp[1] + tblp[...]) * dinv[...]
    t_o[...] = jnp.dot(u, w[...], preferred_element_type=_f32) + bb[0]


def _s5(accf, tblf, dinvf, wt2big, bt2):
    return pl.pallas_call(
        _s5_body,
        grid=(FGRID,),
        in_specs=[_accspec(), _fspec(), _fspec(),
                  _wspec((128, 8)), _wspec((1,))],
        out_specs=[pl.BlockSpec((FBLK, 8), lambda i: (i, 0))],
        out_shape=[jax.ShapeDtypeStruct((NF, 8), _f32)],
    )(accf, tblf, dinvf, wt2big, bt2)[0]


def _row_body(xf, w1, b1, w2, b2, w3, b3, w4, b4, r2_o, tr_o):
    h1 = jnp.maximum(
        jnp.dot(xf[0], w1[...], preferred_element_type=_f32) + b1[...][None, :], 0.0)
    h2 = jnp.maximum(
        jnp.dot(h1, w2[...], preferred_element_type=_f32) + b2[...][None, :], 0.0)
    r2_o[0] = h2
    h3 = jnp.maximum(
        jnp.dot(h2, w3[...], preferred_element_type=_f32) + b3[...][None, :], 0.0)
    tr_o[0] = jnp.dot(h3, w4[...], preferred_element_type=_f32) + b4[0]


def _row(xrf, w1big, be1f, w2big, be2f, w3big, bt1f, wt2big, bt2):
    def bspec(minor):
        return pl.BlockSpec((1, FBLK, minor), lambda b, i: (b, i, 0))

    def wspec(shape):
        return pl.BlockSpec(shape, lambda b, i: tuple(0 for _ in shape))

    return pl.pallas_call(
        _row_body,
        grid=(3, FGRID),
        in_specs=[bspec(128),
                  wspec((128, 128)), wspec((128,)),
                  wspec((128, 128)), wspec((128,)),
                  wspec((128, 128)), wspec((128,)),
                  wspec((128, 8)), wspec((1,))],
        out_specs=[bspec(128), bspec(8)],
        out_shape=[
            jax.ShapeDtypeStruct((3, NF, 128), _f32),
            jax.ShapeDtypeStruct((3, NF, 8), _f32),
        ],
    )(xrf, w1big, be1f, w2big, be2f, w3big, bt1f, wt2big, bt2)


def _pinmlp_body(sf, recip, wp1, bp1, wp2, bp2, out_o):
    m = (sf[0] + sf[1]) * recip[...][None, :]
    g = jnp.dot(m, wp1[...], preferred_element_type=_f32) + bp1[...][None, :]
    g = jnp.maximum(g, 0.0)
    out_o[...] = jnp.dot(g, wp2[...], preferred_element_type=_f32) + bp2[...][None, :]


def _pinmlp(sf, recip, wp1, bp1, wp2, bp2):
    return pl.pallas_call(
        _pinmlp_body,
        out_shape=jax.ShapeDtypeStruct((B, PINS), _f32),
    )(sf, recip, wp1, bp1, wp2, bp2)


def _kron8(w):
    return jnp.kron(jnp.eye(8, dtype=_f32), w.astype(_f32))


def kernel(probe_temperature, probe_locations, cell_adjacency, cell_to_pin_mapping,
           We1, be1, We2, be2, Wp1, bp1, Wp2, bp2, Wt1, bt1, Wt2, bt2):
    epad = (N + jnp.arange(EP - E, dtype=jnp.int32) % (NP - N)).astype(jnp.int32)
    edges = jnp.concatenate(
        [cell_adjacency.astype(jnp.int32),
         jnp.broadcast_to(epad[None], (2, EP - E))],
        axis=1).reshape(2, NW, NCHUNK, 128)

    valid = cell_to_pin_mapping >= 0
    idxp = jnp.where(valid, cell_to_pin_mapping, TRASH).astype(jnp.int32)
    idxw = jnp.concatenate(
        [idxp, jnp.full((NP - N,), TRASH, jnp.int32)]).reshape(8, PCH, 128)
    pww = jnp.concatenate(
        [valid.astype(_f32), jnp.zeros((NP - N,), _f32)]).reshape(8, PCH, 128)

    z16 = jnp.zeros((NP, 16), _f32)
    zp16 = jnp.zeros((PP, 16), _f32)
    zp = jnp.zeros((PP,), _f32)
    ones16 = jnp.ones((128, 16), _f32)

    loc16 = probe_locations.astype(jnp.int32) * 16
    bidx = jnp.arange(B, dtype=jnp.int32)[:, None, None] * (NP * 16)
    cho = jnp.array([0, 1], jnp.int32)[None, None, :]
    flatidx = (bidx + loc16[None, :, None] + cho).reshape(-1)
    flatvals = jnp.stack(
        [probe_temperature, jnp.ones_like(probe_temperature)], axis=-1).reshape(-1)
    xall = jnp.zeros((B * NP * 16,), _f32).at[flatidx].set(flatvals)
    xallf = xall.reshape(B, NF, 128)
    x0f = xallf[0]
    xrf = xallf[1:]

    w1big = _kron8(jnp.zeros((16, 16), _f32).at[:2, :].set(We1))
    w2big = _kron8(We2)
    w3big = _kron8(Wt1)
    wt2big = _kron8(Wt2)
    be1f = jnp.tile(be1, 8)
    be2f = jnp.tile(be2, 8)
    bt1f = jnp.tile(bt1, 8)

    degp = _deg(edges, ones16, z16)
    degf = degp.reshape(2, NF, 128)
    dinvf, tbl1f = _s1(degf, x0f)

    accf1 = _agg(tbl1f.reshape(NP, 16), edges, z16).reshape(2, NF, 128)
    (tbl2f,) = _stage(accf1, tbl1f, dinvf, w1big, be1f)
    accf2 = _agg(tbl2f.reshape(NP, 16), edges, z16).reshape(2, NF, 128)
    tbl3f, h2f = _stage(accf2, tbl2f, dinvf, w2big, be2f, emit_h=True)
    accf3 = _agg(tbl3f.reshape(NP, 16), edges, z16).reshape(2, NF, 128)
    (tbl4f,) = _stage(accf3, tbl3f, dinvf, w3big, bt1f)
    accf4 = _agg(tbl4f.reshape(NP, 16), edges, z16).reshape(2, NF, 128)
    t0f = _s5(accf4, tbl4f, dinvf, wt2big, bt2)

    r2f, trf = _row(xrf, w1big, be1f, w2big, be2f, w3big, bt1f, wt2big, bt2)

    sumsp, cntsp = _pin(h2f.reshape(NP, 16), r2f.reshape(3, NP, 16),
                        idxw, pww, zp16, zp)
    sf = sumsp.reshape(2, B, PP * 16)[:, :, :TRASH * 16]
    counts = cntsp[0] + cntsp[1]
    recip = jnp.repeat(1.0 / jnp.clip(counts[:TRASH], 1.0, None), 16)

    pin_power = _pinmlp(sf, recip, Wp1, bp1, Wp2, bp2)
    temperature = jnp.concatenate(
        [t0f.reshape(NP)[None, :N], trf.reshape(3, NP)[:, :N]], axis=0)
    return pin_power, temperature
